# Initial kernel scaffold; baseline (speedup 1.0000x reference)
import jax
import jax.numpy as jnp
from jax.experimental import pallas as pl


def kernel(x, edge_index, edge_attr, node_weight, batch, edge_attr_batch, l0_w1, l0_b1, l0_w2, l0_b2, l1_w1, l1_b1, l1_w2, l1_b2, out_w1, out_b1, out_w2, out_b2, out_w3, out_b3):
    raise NotImplementedError("write your pallas kernel here")


# speedup vs baseline: 3.6882x; 3.6882x over previous
import functools

import jax
import jax.numpy as jnp
from jax import lax
from jax.experimental import pallas as pl
from jax.experimental.pallas import tpu as pltpu
from jax.experimental.pallas import tpu_sc as plsc

_NC = 2
_NS = 16
_NW = _NC * _NS
_LANES = 16

_SUB = 128
_NSUB_SC = 8
_NSUB_G = 16
_CH_MAX = _SUB * _NSUB_G

_G = 16


def _mesh():
    return plsc.VectorSubcoreMesh(core_axis_name="c", subcore_axis_name="s")


def _sc_scatter_build(e_pad, np_rows, n_stage, d, with_count):
    nsub = _NSUB_SC
    ch = _SUB * nsub
    tpr = np_rows // _NS
    nfull = tpr // ch
    rem = tpr - nfull * ch

    out_types = [jax.ShapeDtypeStruct((_NC, np_rows, d), jnp.float32)]
    if with_count:
        out_types.append(jax.ShapeDtypeStruct((_NC, np_rows, d), jnp.float32))

    def body(src_hbm, idx_hbm, *rest):
        if with_count:
            hout, cout, rows_v, idx_v, acc = rest
        else:
            hout, rows_v, idx_v, acc = rest
            cout = None
        cid = lax.axis_index("c")
        sid = lax.axis_index("s")
        wid = sid * _NC + cid
        base = sid * tpr

        def zero_rows():
            @pl.loop(0, ch)
            def _z(i):
                rows_v[i] = jnp.zeros((_LANES,), jnp.float32)

        def zero_acc():
            for k in range(nfull):
                pltpu.sync_copy(rows_v, acc.at[pl.ds(base + k * ch, ch)])
            if rem:
                pltpu.sync_copy(rows_v.at[pl.ds(0, rem)],
                                acc.at[pl.ds(base + nfull * ch, rem)])

        def drain(dst):
            for k in range(nfull):
                pltpu.sync_copy(acc.at[pl.ds(base + k * ch, ch)], rows_v)
                pltpu.sync_copy(rows_v, dst.at[cid, pl.ds(base + k * ch, ch)])
            if rem:
                pltpu.sync_copy(acc.at[pl.ds(base + nfull * ch, rem)],
                                rows_v.at[pl.ds(0, rem)])
                pltpu.sync_copy(rows_v.at[pl.ds(0, rem)],
                                dst.at[cid, pl.ds(base + nfull * ch, rem)])

        zero_rows()
        zero_acc()
        plsc.subcore_barrier()

        @pl.loop(0, n_stage)
        def _stage(s):
            st = wid * n_stage + s
            pltpu.sync_copy(idx_hbm.at[pl.ds(st * nsub, nsub)], idx_v)
            pltpu.sync_copy(src_hbm.at[pl.ds(st * ch, ch)], rows_v)
            for j in range(nsub):
                pltpu.sync_copy(rows_v.at[pl.ds(j * _SUB, _SUB)],
                                acc.at[idx_v.at[j]], add=True)

        plsc.subcore_barrier()
        drain(hout)

        if with_count:
            plsc.subcore_barrier()
            zero_rows()
            zero_acc()
            plsc.subcore_barrier()

            @pl.loop(0, _SUB)
            def _ones(i):
                rows_v[i] = jnp.ones((_LANES,), jnp.float32)

            @pl.loop(0, n_stage)
            def _stage2(s):
                st = wid * n_stage + s
                pltpu.sync_copy(idx_hbm.at[pl.ds(st * nsub, nsub)], idx_v)
                for j in range(nsub):
                    pltpu.sync_copy(rows_v.at[pl.ds(0, _SUB)],
                                    acc.at[idx_v.at[j]], add=True)

            plsc.subcore_barrier()
            drain(cout)

    return pl.kernel(
        body,
        out_type=tuple(out_types) if with_count else out_types[0],
        mesh=_mesh(),
        compiler_params=pltpu.CompilerParams(use_tc_tiling_on_sc=False),
        scratch_types=[
            pltpu.VMEM((ch, d), jnp.float32),
            pltpu.VMEM((nsub, _SUB), jnp.int32),
            pltpu.VMEM_SHARED((np_rows, d), jnp.float32),
        ],
    )


def _sc_gather_build(e_pad, n_src, n_stage, d):
    nsub = _NSUB_G
    ch = _SUB * nsub

    def body(tab_hbm, idx_hbm, out_hbm, rows_v, idx_v, sem):
        cid = lax.axis_index("c")
        sid = lax.axis_index("s")
        wid = sid * _NC + cid

        @pl.loop(0, n_stage)
        def _stage(s):
            st = wid * n_stage + s
            pltpu.sync_copy(idx_hbm.at[pl.ds(st * nsub, nsub)], idx_v)
            for j in range(nsub):
                pltpu.async_copy(tab_hbm.at[idx_v.at[j]],
                                 rows_v.at[pl.ds(j * _SUB, _SUB)], sem).wait()
            pltpu.sync_copy(rows_v, out_hbm.at[pl.ds(st * ch, ch)])

    return pl.kernel(
        body,
        out_type=jax.ShapeDtypeStruct((e_pad, d), jnp.float32),
        mesh=_mesh(),
        compiler_params=pltpu.CompilerParams(use_tc_tiling_on_sc=False),
        scratch_types=[
            pltpu.VMEM((ch, d), jnp.float32),
            pltpu.VMEM((nsub, _SUB), jnp.int32),
            pltpu.SemaphoreType.DMA,
        ],
    )


def _onehot(ids_col, bn):
    io = lax.broadcasted_iota(jnp.int32, (bn, _G), 1)
    return (ids_col == io).astype(jnp.float32)


def _dotT(a, b):
    return lax.dot_general(a, b, (((0,), (0,)), ((), ())),
                           preferred_element_type=jnp.float32)


def _finalize_build(n, np_rows, bn, d, mode):
    grid = (n // bn,)
    blk_p = pl.BlockSpec((1, bn, d), lambda i: (0, i, 0))
    blk_p1 = pl.BlockSpec((1, bn, d), lambda i: (1, i, 0))
    blk_n = pl.BlockSpec((bn, d), lambda i: (i, 0))
    blk_b = pl.BlockSpec((bn, 1), lambda i: (i, 0))
    blk_g = pl.BlockSpec((_G, d), lambda i: (0, 0))

    def common(i, h, b_ref, gna_ref, ncnt_ref):
        oh = _onehot(b_ref[...], bn)

        @pl.when(i == 0)
        def _():
            gna_ref[...] = jnp.zeros_like(gna_ref)
            if ncnt_ref is not None:
                ncnt_ref[...] = jnp.zeros_like(ncnt_ref)

        gna_ref[...] += _dotT(oh, h)
        if ncnt_ref is not None:
            ncnt_ref[...] += _dotT(oh, jnp.ones((bn, d), jnp.float32))

    if mode == 0:
        def body(hp0, hp1, cp0, cp1, b_ref, h_ref, invc_ref, gna_ref, ncnt_ref):
            i = pl.program_id(0)
            p = hp0[0] + hp1[0]
            c = jnp.maximum(cp0[0] + cp1[0], 1.0)
            invc = 1.0 / c
            h = p * invc
            h_ref[...] = h
            invc_ref[...] = invc
            common(i, h, b_ref, gna_ref, ncnt_ref)

        return pl.pallas_call(
            body,
            grid=grid,
            in_specs=[blk_p, blk_p1, blk_p, blk_p1, blk_b],
            out_specs=[blk_n, blk_n, blk_g, blk_g],
            out_shape=[
                jax.ShapeDtypeStruct((n, d), jnp.float32),
                jax.ShapeDtypeStruct((n, d), jnp.float32),
                jax.ShapeDtypeStruct((_G, d), jnp.float32),
                jax.ShapeDtypeStruct((_G, d), jnp.float32),
            ],
        )

    if mode == 1:
        def body(hp0, hp1, invc_ref, b_ref, h_ref, gna_ref):
            i = pl.program_id(0)
            h = (hp0[0] + hp1[0]) * invc_ref[...]
            h_ref[...] = h
            common(i, h, b_ref, gna_ref, None)

        return pl.pallas_call(
            body,
            grid=grid,
            in_specs=[blk_p, blk_p1, blk_n, blk_b],
            out_specs=[blk_n, blk_g],
            out_shape=[
                jax.ShapeDtypeStruct((n, d), jnp.float32),
                jax.ShapeDtypeStruct((_G, d), jnp.float32),
            ],
        )

    def body(hp0, hp1, invc_ref, b_ref, gna_ref):
        i = pl.program_id(0)
        h = (hp0[0] + hp1[0]) * invc_ref[...]
        common(i, h, b_ref, gna_ref, None)

    return pl.pallas_call(
        body,
        grid=grid,
        in_specs=[blk_p, blk_p1, blk_n, blk_b],
        out_specs=[blk_g],
        out_shape=[jax.ShapeDtypeStruct((_G, d), jnp.float32)],
    )


def _mlp_build(e_pad, be, d, hid, first):
    grid = (e_pad // be,)
    blk_e = pl.BlockSpec((be, d), lambda i: (i, 0))
    blk_b = pl.BlockSpec((be, 1), lambda i: (i, 0))
    blk_w1 = pl.BlockSpec((2 * d, hid), lambda i: (0, 0))
    blk_b1 = pl.BlockSpec((1, hid), lambda i: (0, 0))
    blk_w2 = pl.BlockSpec((hid, d), lambda i: (0, 0))
    blk_b2 = pl.BlockSpec((1, d), lambda i: (0, 0))
    blk_g = pl.BlockSpec((_G, d), lambda i: (0, 0))

    def body(hr_ref, ea_ref, eb_ref, w1_ref, b1_ref, w2_ref, b2_ref, *outs):
        i = pl.program_id(0)
        hr = hr_ref[...]
        ea = ea_ref[...]
        w1 = w1_ref[...]
        z = (jnp.dot(hr, w1[:d], preferred_element_type=jnp.float32)
             + jnp.dot(ea, w1[d:], preferred_element_type=jnp.float32)
             + b1_ref[...])
        z = jnp.maximum(z, 0.0)
        ev = jnp.dot(z, w2_ref[...], preferred_element_type=jnp.float32) + b2_ref[...]
        oh = _onehot(eb_ref[...], be)
        if first:
            e_ref, gea_in_ref, gea_out_ref, ecnt_ref = outs

            @pl.when(i == 0)
            def _():
                gea_in_ref[...] = jnp.zeros_like(gea_in_ref)
                gea_out_ref[...] = jnp.zeros_like(gea_out_ref)
                ecnt_ref[...] = jnp.zeros_like(ecnt_ref)

            gea_in_ref[...] += _dotT(oh, ea)
            gea_out_ref[...] += _dotT(oh, ev)
            ecnt_ref[...] += _dotT(oh, jnp.ones((be, d), jnp.float32))
        else:
            e_ref, gea_out_ref = outs

            @pl.when(i == 0)
            def _():
                gea_out_ref[...] = jnp.zeros_like(gea_out_ref)

            gea_out_ref[...] += _dotT(oh, ev)
        e_ref[...] = ev

    if first:
        out_specs = [blk_e, blk_g, blk_g, blk_g]
        out_shape = [
            jax.ShapeDtypeStruct((e_pad, d), jnp.float32),
            jax.ShapeDtypeStruct((_G, d), jnp.float32),
            jax.ShapeDtypeStruct((_G, d), jnp.float32),
            jax.ShapeDtypeStruct((_G, d), jnp.float32),
        ]
    else:
        out_specs = [blk_e, blk_g]
        out_shape = [
            jax.ShapeDtypeStruct((e_pad, d), jnp.float32),
            jax.ShapeDtypeStruct((_G, d), jnp.float32),
        ]

    return pl.pallas_call(
        body,
        grid=grid,
        in_specs=[blk_e, blk_e, blk_b, blk_w1, blk_b1, blk_w2, blk_b2],
        out_specs=out_specs,
        out_shape=out_shape,
    )


def _final_build(d, hid, out_dim):
    def body(gna0, gna1, gna2, ncnt, gea0, gea1, gea2, ecnt,
             w1, b1, w2, b2, w3, b3, an_ref, ae_ref, o_ref):
        ninv = 1.0 / jnp.maximum(ncnt[...], 1.0)
        einv = 1.0 / jnp.maximum(ecnt[...], 1.0)
        an = jnp.concatenate(
            [gna0[...] * ninv, gna1[...] * ninv, gna2[...] * ninv], axis=1)
        ae = jnp.concatenate(
            [gea0[...] * einv, gea1[...] * einv, gea2[...] * einv], axis=1)
        att = jnp.concatenate([an, ae], axis=1)
        o = jnp.maximum(jnp.dot(att, w1[...], preferred_element_type=jnp.float32)
                        + b1[...], 0.0)
        o = jnp.maximum(jnp.dot(o, w2[...], preferred_element_type=jnp.float32)
                        + b2[...], 0.0)
        o = jnp.dot(o, w3[...], preferred_element_type=jnp.float32) + b3[...]
        an_ref[...] = an
        ae_ref[...] = ae
        o_ref[...] = o

    return pl.pallas_call(
        body,
        out_shape=[
            jax.ShapeDtypeStruct((_G, 3 * d), jnp.float32),
            jax.ShapeDtypeStruct((_G, 3 * d), jnp.float32),
            jax.ShapeDtypeStruct((_G, out_dim), jnp.float32),
        ],
    )


def kernel(x, edge_index, edge_attr, node_weight, batch, edge_attr_batch,
           l0_w1, l0_b1, l0_w2, l0_b2, l1_w1, l1_b1, l1_w2, l1_b2,
           out_w1, out_b1, out_w2, out_b2, out_w3, out_b3):
    n, d = x.shape
    e = edge_index.shape[1]
    hid = l0_w1.shape[1]
    out_dim = out_w3.shape[1]

    blk = _NW * _CH_MAX
    e_pad = -(-e // blk) * blk
    n_stage_sc = e_pad // (_NW * _SUB * _NSUB_SC)
    n_stage_g = e_pad // (_NW * _SUB * _NSUB_G)
    pad = e_pad - e
    np_rows = -(-(n + 1) // (_NS * 8)) * (_NS * 8)

    row = edge_index[0]
    col = edge_index[1]
    colp = jnp.concatenate([col, jnp.full((pad,), n, jnp.int32)])
    rowp = jnp.concatenate([row, jnp.zeros((pad,), jnp.int32)])
    eap = jnp.concatenate([edge_attr, jnp.zeros((pad, d), jnp.float32)])
    ebp = jnp.concatenate([edge_attr_batch, jnp.full((pad,), _G, jnp.int32)])
    col2 = colp.reshape(e_pad // _SUB, _SUB)
    row2 = rowp.reshape(e_pad // _SUB, _SUB)
    eb2 = ebp.reshape(e_pad, 1)
    batch2 = batch.reshape(n, 1)

    bn = 2000
    be = 2048

    scatter_cnt = _sc_scatter_build(e_pad, np_rows, n_stage_sc, d, True)
    scatter = _sc_scatter_build(e_pad, np_rows, n_stage_sc, d, False)
    gather = _sc_gather_build(e_pad, n, n_stage_g, d)
    fin0 = _finalize_build(n, np_rows, bn, d, 0)
    fin1 = _finalize_build(n, np_rows, bn, d, 1)
    fin2 = _finalize_build(n, np_rows, bn, d, 2)
    mlp0 = _mlp_build(e_pad, be, d, hid, True)
    mlp1 = _mlp_build(e_pad, be, d, hid, False)
    final = _final_build(d, hid, out_dim)

    hp, cp = scatter_cnt(eap, col2)
    h0, invc, gna0, ncnt = fin0(hp, hp, cp, cp, batch2)
    hrow0 = gather(h0, row2)
    e0, gea0, gea1, ecnt = mlp0(hrow0, eap, eb2,
                                l0_w1, l0_b1.reshape(1, -1),
                                l0_w2, l0_b2.reshape(1, -1))
    hp1 = scatter(e0, col2)
    h1, gna1 = fin1(hp1, hp1, invc, batch2)
    hrow1 = gather(h1, row2)
    e1, gea2 = mlp1(hrow1, e0, eb2,
                    l1_w1, l1_b1.reshape(1, -1),
                    l1_w2, l1_b2.reshape(1, -1))
    hp2 = scatter(e1, col2)
    (gna2,) = fin2(hp2, hp2, invc, batch2)

    all_node, all_edge, o = final(
        gna0, gna1, gna2, ncnt, gea0, gea1, gea2, ecnt,
        out_w1, out_b1.reshape(1, -1), out_w2, out_b2.reshape(1, -1),
        out_w3, out_b3.reshape(1, -1))
    return (all_node, all_edge, o)


# speedup vs baseline: 4.1075x; 1.1137x over previous
import functools

import jax
import jax.numpy as jnp
from jax import lax
from jax.experimental import pallas as pl
from jax.experimental.pallas import tpu as pltpu
from jax.experimental.pallas import tpu_sc as plsc

_NC = 2
_NS = 16
_NW = _NC * _NS
_LANES = 16

_SUB = 128
_NSUB_SC = 8
_NSUB_G = 16
_CH_MAX = _SUB * _NSUB_G

_G = 16


def _mesh():
    return plsc.VectorSubcoreMesh(core_axis_name="c", subcore_axis_name="s")


def _sc_scatter_build(e_pad, np_rows, n_stage, d, with_count):
    nsub = _NSUB_SC
    ch = _SUB * nsub
    tpr = np_rows // _NS
    nfull = tpr // ch
    rem = tpr - nfull * ch

    out_types = [jax.ShapeDtypeStruct((_NC, np_rows, d), jnp.float32)]
    if with_count:
        out_types.append(jax.ShapeDtypeStruct((_NC, np_rows, d), jnp.float32))

    def body(src_hbm, idx_hbm, gidx_hbm, *rest):
        if with_count:
            hout, cout, rows_v, idx_v, gidx_v, acc, sem = rest
        else:
            hout, rows_v, idx_v, gidx_v, acc, sem = rest
            cout = None
        cid = lax.axis_index("c")
        sid = lax.axis_index("s")
        wid = sid * _NC + cid
        base = sid * tpr

        def zero_rows():
            @pl.loop(0, ch)
            def _z(i):
                rows_v[i] = jnp.zeros((_LANES,), jnp.float32)

        def zero_acc():
            for k in range(nfull):
                pltpu.sync_copy(rows_v, acc.at[pl.ds(base + k * ch, ch)])
            if rem:
                pltpu.sync_copy(rows_v.at[pl.ds(0, rem)],
                                acc.at[pl.ds(base + nfull * ch, rem)])

        def drain(dst):
            for k in range(nfull):
                pltpu.sync_copy(acc.at[pl.ds(base + k * ch, ch)], rows_v)
                pltpu.sync_copy(rows_v, dst.at[cid, pl.ds(base + k * ch, ch)])
            if rem:
                pltpu.sync_copy(acc.at[pl.ds(base + nfull * ch, rem)],
                                rows_v.at[pl.ds(0, rem)])
                pltpu.sync_copy(rows_v.at[pl.ds(0, rem)],
                                dst.at[cid, pl.ds(base + nfull * ch, rem)])

        def scatter_pass(idx2, stage_rows):
            @pl.loop(0, n_stage)
            def _stage(s):
                st = wid * n_stage + s
                i_cp = pltpu.async_copy(idx_hbm.at[pl.ds(st * nsub, nsub)],
                                        idx_v, sem)
                g_cp = pltpu.async_copy(gidx_hbm.at[pl.ds(st * nsub, nsub)],
                                        gidx_v, sem)
                stage_rows(st)
                i_cp.wait()
                g_cp.wait()
                descs = []
                for j in range(nsub):
                    src = rows_v.at[pl.ds(j * _SUB if idx2 else 0, _SUB)]
                    descs.append(pltpu.async_copy(
                        src, acc.at[idx_v.at[j]], sem, add=True))
                    descs.append(pltpu.async_copy(
                        src, acc.at[gidx_v.at[j]], sem, add=True))
                for dsc in descs:
                    dsc.wait()

        zero_rows()
        zero_acc()
        plsc.subcore_barrier()

        def stage_rows_main(st):
            pltpu.sync_copy(src_hbm.at[pl.ds(st * ch, ch)], rows_v)

        scatter_pass(True, stage_rows_main)

        plsc.subcore_barrier()
        drain(hout)

        if with_count:
            plsc.subcore_barrier()
            zero_rows()
            zero_acc()
            plsc.subcore_barrier()

            @pl.loop(0, _SUB)
            def _ones(i):
                rows_v[i] = jnp.ones((_LANES,), jnp.float32)

            scatter_pass(False, lambda st: None)

            plsc.subcore_barrier()
            drain(cout)

    return pl.kernel(
        body,
        out_type=tuple(out_types) if with_count else out_types[0],
        mesh=_mesh(),
        compiler_params=pltpu.CompilerParams(use_tc_tiling_on_sc=False),
        scratch_types=[
            pltpu.VMEM((ch, d), jnp.float32),
            pltpu.VMEM((nsub, _SUB), jnp.int32),
            pltpu.VMEM((nsub, _SUB), jnp.int32),
            pltpu.VMEM_SHARED((np_rows, d), jnp.float32),
            pltpu.SemaphoreType.DMA,
        ],
    )


def _sc_gather_build(e_pad, n_src, n_stage, d):
    nsub = _NSUB_G
    ch = _SUB * nsub

    def body(tab_hbm, idx_hbm, out_hbm, rows_v, idx_v, sem):
        cid = lax.axis_index("c")
        sid = lax.axis_index("s")
        wid = sid * _NC + cid

        @pl.loop(0, n_stage)
        def _stage(s):
            st = wid * n_stage + s
            pltpu.sync_copy(idx_hbm.at[pl.ds(st * nsub, nsub)], idx_v)
            descs = [pltpu.async_copy(tab_hbm.at[idx_v.at[j]],
                                      rows_v.at[pl.ds(j * _SUB, _SUB)], sem)
                     for j in range(nsub)]
            for dsc in descs:
                dsc.wait()
            pltpu.sync_copy(rows_v, out_hbm.at[pl.ds(st * ch, ch)])

    return pl.kernel(
        body,
        out_type=jax.ShapeDtypeStruct((e_pad, d), jnp.float32),
        mesh=_mesh(),
        compiler_params=pltpu.CompilerParams(use_tc_tiling_on_sc=False),
        scratch_types=[
            pltpu.VMEM((ch, d), jnp.float32),
            pltpu.VMEM((nsub, _SUB), jnp.int32),
            pltpu.SemaphoreType.DMA,
        ],
    )


def _onehot_t(ids_row, bn):
    io = lax.broadcasted_iota(jnp.int32, (_G, bn), 0)
    return (ids_row == io).astype(jnp.float32)


def _finalize_build(n, np_rows, bn, d, mode):
    grid = (n // bn,)
    blk_p = pl.BlockSpec((1, bn, d), lambda i: (0, i, 0))
    blk_p1 = pl.BlockSpec((1, bn, d), lambda i: (1, i, 0))
    blk_n = pl.BlockSpec((bn, d), lambda i: (i, 0))
    blk_b = pl.BlockSpec((1, 1, bn), lambda i: (i, 0, 0))
    blk_g = pl.BlockSpec((_G, d), lambda i: (0, 0))

    def common(i, h, b_ref, gna_ref, ncnt_ref):
        oht = _onehot_t(b_ref[0], bn)

        @pl.when(i == 0)
        def _():
            gna_ref[...] = jnp.zeros_like(gna_ref)
            if ncnt_ref is not None:
                ncnt_ref[...] = jnp.zeros_like(ncnt_ref)

        gna_ref[...] += jnp.dot(oht, h, preferred_element_type=jnp.float32)
        if ncnt_ref is not None:
            ncnt_ref[...] += jnp.dot(oht, jnp.ones((bn, d), jnp.float32),
                                     preferred_element_type=jnp.float32)

    if mode == 0:
        def body(hp0, hp1, cp0, cp1, b_ref, h_ref, invc_ref, gna_ref, ncnt_ref):
            i = pl.program_id(0)
            p = hp0[0] + hp1[0]
            c = jnp.maximum(cp0[0] + cp1[0], 1.0)
            invc = 1.0 / c
            h = p * invc
            h_ref[...] = h
            invc_ref[...] = invc
            common(i, h, b_ref, gna_ref, ncnt_ref)

        return pl.pallas_call(
            body,
            grid=grid,
            in_specs=[blk_p, blk_p1, blk_p, blk_p1, blk_b],
            out_specs=[blk_n, blk_n, blk_g, blk_g],
            out_shape=[
                jax.ShapeDtypeStruct((n, d), jnp.float32),
                jax.ShapeDtypeStruct((n, d), jnp.float32),
                jax.ShapeDtypeStruct((_G, d), jnp.float32),
                jax.ShapeDtypeStruct((_G, d), jnp.float32),
            ],
        )

    if mode == 1:
        def body(hp0, hp1, invc_ref, b_ref, h_ref, gna_ref):
            i = pl.program_id(0)
            h = (hp0[0] + hp1[0]) * invc_ref[...]
            h_ref[...] = h
            common(i, h, b_ref, gna_ref, None)

        return pl.pallas_call(
            body,
            grid=grid,
            in_specs=[blk_p, blk_p1, blk_n, blk_b],
            out_specs=[blk_n, blk_g],
            out_shape=[
                jax.ShapeDtypeStruct((n, d), jnp.float32),
                jax.ShapeDtypeStruct((_G, d), jnp.float32),
            ],
        )

    def body(hp0, hp1, invc_ref, b_ref, gna_ref):
        i = pl.program_id(0)
        h = (hp0[0] + hp1[0]) * invc_ref[...]
        common(i, h, b_ref, gna_ref, None)

    return pl.pallas_call(
        body,
        grid=grid,
        in_specs=[blk_p, blk_p1, blk_n, blk_b],
        out_specs=[blk_g],
        out_shape=[jax.ShapeDtypeStruct((_G, d), jnp.float32)],
    )


def _mlp_build(e_pad, be, d, hid):
    grid = (e_pad // be,)
    blk_e = pl.BlockSpec((be, d), lambda i: (i, 0))
    blk_w1 = pl.BlockSpec((2 * d, hid), lambda i: (0, 0))
    blk_b1 = pl.BlockSpec((1, hid), lambda i: (0, 0))
    blk_w2 = pl.BlockSpec((hid, d), lambda i: (0, 0))
    blk_b2 = pl.BlockSpec((1, d), lambda i: (0, 0))

    def body(hr_ref, ea_ref, w1_ref, b1_ref, w2_ref, b2_ref, e_ref):
        w1 = w1_ref[...]
        z = (jnp.dot(hr_ref[...], w1[:d], preferred_element_type=jnp.float32)
             + jnp.dot(ea_ref[...], w1[d:], preferred_element_type=jnp.float32)
             + b1_ref[...])
        z = jnp.maximum(z, 0.0)
        e_ref[...] = (jnp.dot(z, w2_ref[...], preferred_element_type=jnp.float32)
                      + b2_ref[...])

    return pl.pallas_call(
        body,
        grid=grid,
        in_specs=[blk_e, blk_e, blk_w1, blk_b1, blk_w2, blk_b2],
        out_specs=blk_e,
        out_shape=jax.ShapeDtypeStruct((e_pad, d), jnp.float32),
    )


def _final_build(d, hid, out_dim):
    def body(gna0, gna1, gna2, ncnt, gea0, gea1, gea2, ecnt,
             w1, b1, w2, b2, w3, b3, an_ref, ae_ref, o_ref):
        ninv = 1.0 / jnp.maximum(ncnt[...], 1.0)
        einv = 1.0 / jnp.maximum(ecnt[0] + ecnt[1], 1.0)
        an = jnp.concatenate(
            [gna0[...] * ninv, gna1[...] * ninv, gna2[...] * ninv], axis=1)
        ae = jnp.concatenate(
            [(gea0[0] + gea0[1]) * einv, (gea1[0] + gea1[1]) * einv,
             (gea2[0] + gea2[1]) * einv], axis=1)
        att = jnp.concatenate([an, ae], axis=1)
        o = jnp.maximum(jnp.dot(att, w1[...], preferred_element_type=jnp.float32)
                        + b1[...], 0.0)
        o = jnp.maximum(jnp.dot(o, w2[...], preferred_element_type=jnp.float32)
                        + b2[...], 0.0)
        o = jnp.dot(o, w3[...], preferred_element_type=jnp.float32) + b3[...]
        an_ref[...] = an
        ae_ref[...] = ae
        o_ref[...] = o

    return pl.pallas_call(
        body,
        out_shape=[
            jax.ShapeDtypeStruct((_G, 3 * d), jnp.float32),
            jax.ShapeDtypeStruct((_G, 3 * d), jnp.float32),
            jax.ShapeDtypeStruct((_G, out_dim), jnp.float32),
        ],
    )


def kernel(x, edge_index, edge_attr, node_weight, batch, edge_attr_batch,
           l0_w1, l0_b1, l0_w2, l0_b2, l1_w1, l1_b1, l1_w2, l1_b2,
           out_w1, out_b1, out_w2, out_b2, out_w3, out_b3):
    n, d = x.shape
    e = edge_index.shape[1]
    hid = l0_w1.shape[1]
    out_dim = out_w3.shape[1]

    blk = _NW * _CH_MAX
    e_pad = -(-e // blk) * blk
    n_stage_sc = e_pad // (_NW * _SUB * _NSUB_SC)
    n_stage_g = e_pad // (_NW * _SUB * _NSUB_G)
    pad = e_pad - e
    pool_base = n + 8
    np_rows = -(-(pool_base + _G + 1) // (_NS * 8)) * (_NS * 8)

    row = edge_index[0]
    col = edge_index[1]
    colp = jnp.concatenate([col, jnp.full((pad,), n, jnp.int32)])
    rowp = jnp.concatenate([row, jnp.zeros((pad,), jnp.int32)])
    eap = jnp.concatenate([edge_attr, jnp.zeros((pad, d), jnp.float32)])
    ebp = jnp.concatenate([edge_attr_batch + pool_base,
                           jnp.full((pad,), pool_base + _G, jnp.int32)])
    col2 = colp.reshape(e_pad // _SUB, _SUB)
    row2 = rowp.reshape(e_pad // _SUB, _SUB)
    gb2 = ebp.reshape(e_pad // _SUB, _SUB)
    bn = 2000
    be = 2048
    batch2 = batch.reshape(n // bn, 1, bn)

    scatter_cnt = _sc_scatter_build(e_pad, np_rows, n_stage_sc, d, True)
    scatter = _sc_scatter_build(e_pad, np_rows, n_stage_sc, d, False)
    gather = _sc_gather_build(e_pad, n, n_stage_g, d)
    fin0 = _finalize_build(n, np_rows, bn, d, 0)
    fin1 = _finalize_build(n, np_rows, bn, d, 1)
    fin2 = _finalize_build(n, np_rows, bn, d, 2)
    mlp = _mlp_build(e_pad, be, d, hid)
    final = _final_build(d, hid, out_dim)

    hp, cp = scatter_cnt(eap, col2, gb2)
    h0, invc, gna0, ncnt = fin0(hp, hp, cp, cp, batch2)
    hrow0 = gather(h0, row2)
    e0 = mlp(hrow0, eap, l0_w1, l0_b1.reshape(1, -1),
             l0_w2, l0_b2.reshape(1, -1))
    hp1 = scatter(e0, col2, gb2)
    h1, gna1 = fin1(hp1, hp1, invc, batch2)
    hrow1 = gather(h1, row2)
    e1 = mlp(hrow1, e0, l1_w1, l1_b1.reshape(1, -1),
             l1_w2, l1_b2.reshape(1, -1))
    hp2 = scatter(e1, col2, gb2)
    (gna2,) = fin2(hp2, hp2, invc, batch2)

    gea0 = lax.slice(hp, (0, pool_base, 0), (2, pool_base + _G, d))
    gea1 = lax.slice(hp1, (0, pool_base, 0), (2, pool_base + _G, d))
    gea2 = lax.slice(hp2, (0, pool_base, 0), (2, pool_base + _G, d))
    ecnt = lax.slice(cp, (0, pool_base, 0), (2, pool_base + _G, d))

    all_node, all_edge, o = final(
        gna0, gna1, gna2, ncnt, gea0, gea1, gea2, ecnt,
        out_w1, out_b1.reshape(1, -1), out_w2, out_b2.reshape(1, -1),
        out_w3, out_b3.reshape(1, -1))
    return (all_node, all_edge, o)


# speedup vs baseline: 8.1469x; 1.9834x over previous
import functools

import jax
import jax.numpy as jnp
from jax import lax
from jax.experimental import pallas as pl
from jax.experimental.pallas import tpu as pltpu
from jax.experimental.pallas import tpu_sc as plsc

_NC = 2
_NS = 16
_NW = _NC * _NS
_LANES = 16

_SUB = 128
_NSUB_SC = 8
_NSUB_G = 16
_CH_MAX = _SUB * _NSUB_G

_G = 16


def _mesh():
    return plsc.VectorSubcoreMesh(core_axis_name="c", subcore_axis_name="s")


def _sc_scatter_build(e_pad, np_rows, n_stage, d, with_count):
    nsub = _NSUB_SC
    ch = _SUB * nsub
    tpr = np_rows // _NS
    nfull = tpr // ch
    rem = tpr - nfull * ch

    out_types = [jax.ShapeDtypeStruct((_NC, np_rows, d), jnp.float32)]
    if with_count:
        out_types.append(jax.ShapeDtypeStruct((_NC, np_rows, d), jnp.float32))

    def body(src_hbm, idx_hbm, gidx_hbm, *rest):
        if with_count:
            hout, cout, rows_v, idx_v, gidx_v, acc, sem = rest
        else:
            hout, rows_v, idx_v, gidx_v, acc, sem = rest
            cout = None
        cid = lax.axis_index("c")
        sid = lax.axis_index("s")
        wid = sid * _NC + cid
        base = sid * tpr

        rows_r = rows_v

        def zero_rows():
            @pl.loop(0, ch)
            def _z(i):
                rows_v[i] = jnp.zeros((_LANES,), jnp.float32)

        def zero_acc():
            for k in range(nfull):
                pltpu.sync_copy(rows_r, acc.at[pl.ds(base + k * ch, ch)])
            if rem:
                pltpu.sync_copy(rows_r.at[pl.ds(0, rem)],
                                acc.at[pl.ds(base + nfull * ch, rem)])

        def drain(dst):
            for k in range(nfull):
                pltpu.sync_copy(acc.at[pl.ds(base + k * ch, ch)], rows_r)
                pltpu.sync_copy(rows_r, dst.at[cid, pl.ds(base + k * ch, ch)])
            if rem:
                pltpu.sync_copy(acc.at[pl.ds(base + nfull * ch, rem)],
                                rows_r.at[pl.ds(0, rem)])
                pltpu.sync_copy(rows_r.at[pl.ds(0, rem)],
                                dst.at[cid, pl.ds(base + nfull * ch, rem)])

        def scatter_pass(idx2, stage_rows):
            @pl.loop(0, n_stage)
            def _stage(s):
                st = wid * n_stage + s
                i_cp = pltpu.async_copy(idx_hbm.at[pl.ds(st * nsub, nsub)],
                                        idx_v, sem)
                g_cp = pltpu.async_copy(gidx_hbm.at[pl.ds(st * nsub, nsub)],
                                        gidx_v, sem)
                stage_rows(st)
                i_cp.wait()
                g_cp.wait()
                descs = []
                for j in range(nsub):
                    src = rows_r.at[pl.ds(j * _SUB if idx2 else 0, _SUB)]
                    descs.append(pltpu.async_copy(
                        src, acc.at[idx_v.at[j]], sem, add=True))
                    descs.append(pltpu.async_copy(
                        src, acc.at[gidx_v.at[j]], sem, add=True))
                for dsc in descs:
                    dsc.wait()

        zero_rows()
        zero_acc()
        plsc.subcore_barrier()

        def stage_rows_main(st):
            pltpu.sync_copy(src_hbm.at[pl.ds(st * ch, ch)], rows_v)

        scatter_pass(True, stage_rows_main)

        plsc.subcore_barrier()
        drain(hout)

        if with_count:
            plsc.subcore_barrier()
            zero_rows()
            zero_acc()
            plsc.subcore_barrier()

            @pl.loop(0, _SUB)
            def _ones(i):
                rows_v[i] = jnp.ones((_LANES,), jnp.float32)

            scatter_pass(False, lambda st: None)

            plsc.subcore_barrier()
            drain(cout)

    return pl.kernel(
        body,
        out_type=tuple(out_types) if with_count else out_types[0],
        mesh=_mesh(),
        compiler_params=pltpu.CompilerParams(use_tc_tiling_on_sc=False),
        scratch_types=[
            pltpu.VMEM((ch, d), jnp.float32),
            pltpu.VMEM((nsub, _SUB), jnp.int32),
            pltpu.VMEM((nsub, _SUB), jnp.int32),
            pltpu.VMEM_SHARED((np_rows, d), jnp.float32),
            pltpu.SemaphoreType.DMA,
        ],
    )


def _sc_gather_build(e_pad, n_src, n_stage, d):
    nsub = _NSUB_G
    ch = _SUB * nsub

    def body(tab_hbm, idx_hbm, out_hbm, rows_v, idx_v, sem):
        cid = lax.axis_index("c")
        sid = lax.axis_index("s")
        wid = sid * _NC + cid

        @pl.loop(0, n_stage)
        def _stage(s):
            st = wid * n_stage + s
            pltpu.sync_copy(idx_hbm.at[pl.ds(st * nsub, nsub)], idx_v)
            descs = [pltpu.async_copy(tab_hbm.at[idx_v.at[j]],
                                      rows_v.at[pl.ds(j * _SUB, _SUB)], sem)
                     for j in range(nsub)]
            for dsc in descs:
                dsc.wait()
            pltpu.sync_copy(rows_v, out_hbm.at[pl.ds(st * ch, ch)])

    return pl.kernel(
        body,
        out_type=jax.ShapeDtypeStruct((e_pad, d), jnp.float32),
        mesh=_mesh(),
        compiler_params=pltpu.CompilerParams(use_tc_tiling_on_sc=False),
        scratch_types=[
            pltpu.VMEM((ch, d), jnp.float32),
            pltpu.VMEM((nsub, _SUB), jnp.int32),
            pltpu.SemaphoreType.DMA,
        ],
    )


def _onehot_t(ids_row, bn):
    io = lax.broadcasted_iota(jnp.int32, (_G, bn), 0)
    return (ids_row == io).astype(jnp.float32)


def _finalize_build(n, np_rows, bn, d, mode):
    grid = (n // bn,)
    blk_p = pl.BlockSpec((1, bn, d), lambda i: (0, i, 0))
    blk_p1 = pl.BlockSpec((1, bn, d), lambda i: (1, i, 0))
    blk_n = pl.BlockSpec((bn, d), lambda i: (i, 0))
    blk_b = pl.BlockSpec((1, 1, bn), lambda i: (i, 0, 0))
    blk_g = pl.BlockSpec((_G, d), lambda i: (0, 0))

    def common(i, h, b_ref, gna_ref, ncnt_ref):
        oht = _onehot_t(b_ref[0], bn)

        @pl.when(i == 0)
        def _():
            gna_ref[...] = jnp.zeros_like(gna_ref)
            if ncnt_ref is not None:
                ncnt_ref[...] = jnp.zeros_like(ncnt_ref)

        gna_ref[...] += jnp.dot(oht, h, preferred_element_type=jnp.float32)
        if ncnt_ref is not None:
            ncnt_ref[...] += jnp.dot(oht, jnp.ones((bn, d), jnp.float32),
                                     preferred_element_type=jnp.float32)

    if mode == 0:
        def body(hp0, hp1, cp0, cp1, b_ref, h_ref, invc_ref, gna_ref, ncnt_ref):
            i = pl.program_id(0)
            p = hp0[0] + hp1[0]
            c = jnp.maximum(cp0[0] + cp1[0], 1.0)
            invc = 1.0 / c
            h = p * invc
            h_ref[...] = h
            invc_ref[...] = invc
            common(i, h, b_ref, gna_ref, ncnt_ref)

        return pl.pallas_call(
            body,
            grid=grid,
            in_specs=[blk_p, blk_p1, blk_p, blk_p1, blk_b],
            out_specs=[blk_n, blk_n, blk_g, blk_g],
            out_shape=[
                jax.ShapeDtypeStruct((n, d), jnp.float32),
                jax.ShapeDtypeStruct((n, d), jnp.float32),
                jax.ShapeDtypeStruct((_G, d), jnp.float32),
                jax.ShapeDtypeStruct((_G, d), jnp.float32),
            ],
        )

    if mode == 1:
        def body(hp0, hp1, invc_ref, b_ref, h_ref, gna_ref):
            i = pl.program_id(0)
            h = (hp0[0] + hp1[0]) * invc_ref[...]
            h_ref[...] = h
            common(i, h, b_ref, gna_ref, None)

        return pl.pallas_call(
            body,
            grid=grid,
            in_specs=[blk_p, blk_p1, blk_n, blk_b],
            out_specs=[blk_n, blk_g],
            out_shape=[
                jax.ShapeDtypeStruct((n, d), jnp.float32),
                jax.ShapeDtypeStruct((_G, d), jnp.float32),
            ],
        )

    def body(hp0, hp1, invc_ref, b_ref, gna_ref):
        i = pl.program_id(0)
        h = (hp0[0] + hp1[0]) * invc_ref[...]
        common(i, h, b_ref, gna_ref, None)

    return pl.pallas_call(
        body,
        grid=grid,
        in_specs=[blk_p, blk_p1, blk_n, blk_b],
        out_specs=[blk_g],
        out_shape=[jax.ShapeDtypeStruct((_G, d), jnp.float32)],
    )


def _mk_bd(w, in_step, out_step, reps, in_base, out_base):
    m = jnp.zeros((128, 128), jnp.float32)
    for t in range(reps):
        m = m.at[in_base + t * in_step:in_base + t * in_step + w.shape[0],
                 out_base + t * out_step:out_base + t * out_step + w.shape[1]
                 ].set(w)
    return m


def _mlp_build(e_pad, bep, d, hid):
    rows = e_pad // 8
    grid = (rows // bep,)
    blk_e = pl.BlockSpec((bep, 128), lambda i: (i, 0))
    blk_m = pl.BlockSpec((128, 128), lambda i: (0, 0))
    blk_b = pl.BlockSpec((1, 128), lambda i: (0, 0))

    def body(hr_ref, ea_ref, mae, mao, mbe, mbo, m2e, m2o, b1p, b2p, e_ref):
        hr = hr_ref[...]
        ea = ea_ref[...]
        ze = (jnp.dot(hr, mae[...], preferred_element_type=jnp.float32)
              + jnp.dot(ea, mbe[...], preferred_element_type=jnp.float32)
              + b1p[...])
        zo = (jnp.dot(hr, mao[...], preferred_element_type=jnp.float32)
              + jnp.dot(ea, mbo[...], preferred_element_type=jnp.float32)
              + b1p[...])
        e_ref[...] = (
            jnp.dot(jnp.maximum(ze, 0.0), m2e[...],
                    preferred_element_type=jnp.float32)
            + jnp.dot(jnp.maximum(zo, 0.0), m2o[...],
                      preferred_element_type=jnp.float32)
            + b2p[...])

    return pl.pallas_call(
        body,
        grid=grid,
        in_specs=[blk_e, blk_e] + [blk_m] * 6 + [blk_b] * 2,
        out_specs=blk_e,
        out_shape=jax.ShapeDtypeStruct((rows, 128), jnp.float32),
    )


def _mlp_mats(w1, b1, w2, b2, d):
    w1a, w1b = w1[:d], w1[d:]
    return (_mk_bd(w1a, 16, 32, 4, 0, 0), _mk_bd(w1a, 16, 32, 4, 64, 0),
            _mk_bd(w1b, 16, 32, 4, 0, 0), _mk_bd(w1b, 16, 32, 4, 64, 0),
            _mk_bd(w2, 32, 16, 4, 0, 0), _mk_bd(w2, 32, 16, 4, 0, 64),
            jnp.tile(b1, 4).reshape(1, 128), jnp.tile(b2, 8).reshape(1, 128))


def _final_build(d, hid, out_dim):
    def body(gna0, gna1, gna2, ncnt, gea0, gea1, gea2, ecnt,
             w1, b1, w2, b2, w3, b3, an_ref, ae_ref, o_ref):
        ninv = 1.0 / jnp.maximum(ncnt[...], 1.0)
        einv = 1.0 / jnp.maximum(ecnt[0] + ecnt[1], 1.0)
        an = jnp.concatenate(
            [gna0[...] * ninv, gna1[...] * ninv, gna2[...] * ninv], axis=1)
        ae = jnp.concatenate(
            [(gea0[0] + gea0[1]) * einv, (gea1[0] + gea1[1]) * einv,
             (gea2[0] + gea2[1]) * einv], axis=1)
        att = jnp.concatenate([an, ae], axis=1)
        o = jnp.maximum(jnp.dot(att, w1[...], preferred_element_type=jnp.float32)
                        + b1[...], 0.0)
        o = jnp.maximum(jnp.dot(o, w2[...], preferred_element_type=jnp.float32)
                        + b2[...], 0.0)
        o = jnp.dot(o, w3[...], preferred_element_type=jnp.float32) + b3[...]
        an_ref[...] = an
        ae_ref[...] = ae
        o_ref[...] = o

    return pl.pallas_call(
        body,
        out_shape=[
            jax.ShapeDtypeStruct((_G, 3 * d), jnp.float32),
            jax.ShapeDtypeStruct((_G, 3 * d), jnp.float32),
            jax.ShapeDtypeStruct((_G, out_dim), jnp.float32),
        ],
    )


def kernel(x, edge_index, edge_attr, node_weight, batch, edge_attr_batch,
           l0_w1, l0_b1, l0_w2, l0_b2, l1_w1, l1_b1, l1_w2, l1_b2,
           out_w1, out_b1, out_w2, out_b2, out_w3, out_b3):
    n, d = x.shape
    e = edge_index.shape[1]
    hid = l0_w1.shape[1]
    out_dim = out_w3.shape[1]

    blk = _NW * _CH_MAX
    e_pad = -(-e // blk) * blk
    n_stage_sc = e_pad // (_NW * _SUB * _NSUB_SC)
    n_stage_g = e_pad // (_NW * _SUB * _NSUB_G)
    pad = e_pad - e
    pool_base = n + 8
    np_rows = -(-(pool_base + _G + 1) // (_NS * 8)) * (_NS * 8)

    row = edge_index[0]
    col = edge_index[1]
    colp = jnp.concatenate([col, jnp.full((pad,), n, jnp.int32)])
    rowp = jnp.concatenate([row, jnp.zeros((pad,), jnp.int32)])
    eap = jnp.concatenate([edge_attr, jnp.zeros((pad, d), jnp.float32)])
    ebp = jnp.concatenate([edge_attr_batch + pool_base,
                           jnp.full((pad,), pool_base + _G, jnp.int32)])
    col2 = colp.reshape(e_pad // _SUB, _SUB)
    row2 = rowp.reshape(e_pad // _SUB, _SUB)
    gb2 = ebp.reshape(e_pad // _SUB, _SUB)
    bn = 2000
    bep = 4096
    batch2 = batch.reshape(n // bn, 1, bn)

    scatter_cnt = _sc_scatter_build(e_pad, np_rows, n_stage_sc, d, True)
    scatter = _sc_scatter_build(e_pad, np_rows, n_stage_sc, d, False)
    gather = _sc_gather_build(e_pad, n, n_stage_g, d)
    fin0 = _finalize_build(n, np_rows, bn, d, 0)
    fin1 = _finalize_build(n, np_rows, bn, d, 1)
    fin2 = _finalize_build(n, np_rows, bn, d, 2)
    mlp = _mlp_build(e_pad, bep, d, hid)
    final = _final_build(d, hid, out_dim)

    eap_p = eap.reshape(e_pad // 8, 8 * d)

    hp, cp = scatter_cnt(eap, col2, gb2)
    h0, invc, gna0, ncnt = fin0(hp, hp, cp, cp, batch2)
    hrow0 = gather(h0, row2)
    e0_p = mlp(hrow0.reshape(e_pad // 8, 8 * d), eap_p,
               *_mlp_mats(l0_w1, l0_b1, l0_w2, l0_b2, d))
    e0 = e0_p.reshape(e_pad, d)
    hp1 = scatter(e0, col2, gb2)
    h1, gna1 = fin1(hp1, hp1, invc, batch2)
    hrow1 = gather(h1, row2)
    e1 = mlp(hrow1.reshape(e_pad // 8, 8 * d), e0_p,
             *_mlp_mats(l1_w1, l1_b1, l1_w2, l1_b2, d)).reshape(e_pad, d)
    hp2 = scatter(e1, col2, gb2)
    (gna2,) = fin2(hp2, hp2, invc, batch2)

    gea0 = lax.slice(hp, (0, pool_base, 0), (2, pool_base + _G, d))
    gea1 = lax.slice(hp1, (0, pool_base, 0), (2, pool_base + _G, d))
    gea2 = lax.slice(hp2, (0, pool_base, 0), (2, pool_base + _G, d))
    ecnt = lax.slice(cp, (0, pool_base, 0), (2, pool_base + _G, d))

    all_node, all_edge, o = final(
        gna0, gna1, gna2, ncnt, gea0, gea1, gea2, ecnt,
        out_w1, out_b1.reshape(1, -1), out_w2, out_b2.reshape(1, -1),
        out_w3, out_b3.reshape(1, -1))
    return (all_node, all_edge, o)


# speedup vs baseline: 10.3168x; 1.2663x over previous
import functools

import jax
import jax.numpy as jnp
from jax import lax
from jax.experimental import pallas as pl
from jax.experimental.pallas import tpu as pltpu
from jax.experimental.pallas import tpu_sc as plsc

_NC = 2
_NS = 16
_NW = _NC * _NS
_LANES = 16

_SUB = 128
_NSUB_SC = 10
_NSUB_G = 20
_CH_MAX = _SUB * _NSUB_G * 2

_G = 16


def _mesh():
    return plsc.VectorSubcoreMesh(core_axis_name="c", subcore_axis_name="s")


def _sc_scatter_build(e_pad, np_rows, n_stage, d, with_count):
    nsub = _NSUB_SC
    ch = _SUB * nsub
    tpr = np_rows // _NS
    nfull = tpr // ch
    rem = tpr - nfull * ch

    out_types = [jax.ShapeDtypeStruct((_NC, np_rows, d), jnp.float32)]
    if with_count:
        out_types.append(jax.ShapeDtypeStruct((_NC, np_rows, d), jnp.float32))

    def body(src_hbm, idx_hbm, gidx_hbm, *rest):
        if with_count:
            hout, cout, rows_v, idx_v, gidx_v, acc, sem = rest
        else:
            hout, rows_v, idx_v, gidx_v, acc, sem = rest
            cout = None
        cid = lax.axis_index("c")
        sid = lax.axis_index("s")
        wid = sid * _NC + cid
        base = sid * tpr

        rows_r = rows_v

        def zero_rows():
            @pl.loop(0, ch)
            def _z(i):
                rows_v[i] = jnp.zeros((_LANES,), jnp.float32)

        def zero_acc():
            for k in range(nfull):
                pltpu.sync_copy(rows_r, acc.at[pl.ds(base + k * ch, ch)])
            if rem:
                pltpu.sync_copy(rows_r.at[pl.ds(0, rem)],
                                acc.at[pl.ds(base + nfull * ch, rem)])

        def drain(dst):
            for k in range(nfull):
                pltpu.sync_copy(acc.at[pl.ds(base + k * ch, ch)], rows_r)
                pltpu.sync_copy(rows_r, dst.at[cid, pl.ds(base + k * ch, ch)])
            if rem:
                pltpu.sync_copy(acc.at[pl.ds(base + nfull * ch, rem)],
                                rows_r.at[pl.ds(0, rem)])
                pltpu.sync_copy(rows_r.at[pl.ds(0, rem)],
                                dst.at[cid, pl.ds(base + nfull * ch, rem)])

        def scatter_pass(idx2, stage_rows):
            @pl.loop(0, n_stage)
            def _stage(s):
                st = wid * n_stage + s
                i_cp = pltpu.async_copy(idx_hbm.at[pl.ds(st * nsub, nsub)],
                                        idx_v, sem)
                g_cp = pltpu.async_copy(gidx_hbm.at[pl.ds(st * nsub, nsub)],
                                        gidx_v, sem)
                stage_rows(st)
                i_cp.wait()
                g_cp.wait()
                descs = []
                for j in range(nsub):
                    src = rows_r.at[pl.ds(j * _SUB if idx2 else 0, _SUB)]
                    descs.append(pltpu.async_copy(
                        src, acc.at[idx_v.at[j]], sem, add=True))
                    descs.append(pltpu.async_copy(
                        src, acc.at[gidx_v.at[j]], sem, add=True))
                for dsc in descs:
                    dsc.wait()

        zero_rows()
        zero_acc()
        plsc.subcore_barrier()

        def stage_rows_main(st):
            pltpu.sync_copy(src_hbm.at[pl.ds(st * ch, ch)], rows_v)

        scatter_pass(True, stage_rows_main)

        plsc.subcore_barrier()
        drain(hout)

        if with_count:
            plsc.subcore_barrier()
            zero_rows()
            zero_acc()
            plsc.subcore_barrier()

            @pl.loop(0, _SUB)
            def _ones(i):
                rows_v[i] = jnp.ones((_LANES,), jnp.float32)

            scatter_pass(False, lambda st: None)

            plsc.subcore_barrier()
            drain(cout)

    return pl.kernel(
        body,
        out_type=tuple(out_types) if with_count else out_types[0],
        mesh=_mesh(),
        compiler_params=pltpu.CompilerParams(use_tc_tiling_on_sc=False),
        scratch_types=[
            pltpu.VMEM((ch, d), jnp.float32),
            pltpu.VMEM((nsub, _SUB), jnp.int32),
            pltpu.VMEM((nsub, _SUB), jnp.int32),
            pltpu.VMEM_SHARED((np_rows, d), jnp.float32),
            pltpu.SemaphoreType.DMA,
        ],
    )


def _sc_gather_build(e_pad, n_src, n_stage, d):
    nsub = _NSUB_G
    ch = _SUB * nsub

    def body(tab_hbm, idx_hbm, out_hbm, rows_v, idx_v, sem):
        cid = lax.axis_index("c")
        sid = lax.axis_index("s")
        wid = sid * _NC + cid

        @pl.loop(0, n_stage)
        def _stage(s):
            st = wid * n_stage + s
            pltpu.sync_copy(idx_hbm.at[pl.ds(st * nsub, nsub)], idx_v)
            descs = [pltpu.async_copy(tab_hbm.at[idx_v.at[j]],
                                      rows_v.at[pl.ds(j * _SUB, _SUB)], sem)
                     for j in range(nsub)]
            for dsc in descs:
                dsc.wait()
            pltpu.sync_copy(rows_v, out_hbm.at[pl.ds(st * ch, ch)])

    return pl.kernel(
        body,
        out_type=jax.ShapeDtypeStruct((e_pad, d), jnp.float32),
        mesh=_mesh(),
        compiler_params=pltpu.CompilerParams(use_tc_tiling_on_sc=False),
        scratch_types=[
            pltpu.VMEM((ch, d), jnp.float32),
            pltpu.VMEM((nsub, _SUB), jnp.int32),
            pltpu.SemaphoreType.DMA,
        ],
    )


def _onehot_t(ids_row, bn):
    io = lax.broadcasted_iota(jnp.int32, (_G, bn), 0)
    return (ids_row == io).astype(jnp.float32)


def _finalize_build(n, np_rows, bn, d, mode):
    grid = (n // bn,)
    blk_p = pl.BlockSpec((1, bn, d), lambda i: (0, i, 0))
    blk_p1 = pl.BlockSpec((1, bn, d), lambda i: (1, i, 0))
    blk_n = pl.BlockSpec((bn, d), lambda i: (i, 0))
    blk_b = pl.BlockSpec((1, 1, bn), lambda i: (i, 0, 0))
    blk_g = pl.BlockSpec((_G, d), lambda i: (0, 0))

    def common(i, h, b_ref, gna_ref, ncnt_ref):
        oht = _onehot_t(b_ref[0], bn)

        @pl.when(i == 0)
        def _():
            gna_ref[...] = jnp.zeros_like(gna_ref)
            if ncnt_ref is not None:
                ncnt_ref[...] = jnp.zeros_like(ncnt_ref)

        gna_ref[...] += jnp.dot(oht, h, preferred_element_type=jnp.float32)
        if ncnt_ref is not None:
            ncnt_ref[...] += jnp.dot(oht, jnp.ones((bn, d), jnp.float32),
                                     preferred_element_type=jnp.float32)

    if mode == 0:
        def body(hp0, hp1, cp0, cp1, b_ref, h_ref, invc_ref, gna_ref, ncnt_ref):
            i = pl.program_id(0)
            p = hp0[0] + hp1[0]
            c = jnp.maximum(cp0[0] + cp1[0], 1.0)
            invc = 1.0 / c
            h = p * invc
            h_ref[...] = h
            invc_ref[...] = invc
            common(i, h, b_ref, gna_ref, ncnt_ref)

        return pl.pallas_call(
            body,
            grid=grid,
            in_specs=[blk_p, blk_p1, blk_p, blk_p1, blk_b],
            out_specs=[blk_n, blk_n, blk_g, blk_g],
            out_shape=[
                jax.ShapeDtypeStruct((n, d), jnp.float32),
                jax.ShapeDtypeStruct((n, d), jnp.float32),
                jax.ShapeDtypeStruct((_G, d), jnp.float32),
                jax.ShapeDtypeStruct((_G, d), jnp.float32),
            ],
        )

    if mode == 1:
        def body(hp0, hp1, invc_ref, b_ref, h_ref, gna_ref):
            i = pl.program_id(0)
            h = (hp0[0] + hp1[0]) * invc_ref[...]
            h_ref[...] = h
            common(i, h, b_ref, gna_ref, None)

        return pl.pallas_call(
            body,
            grid=grid,
            in_specs=[blk_p, blk_p1, blk_n, blk_b],
            out_specs=[blk_n, blk_g],
            out_shape=[
                jax.ShapeDtypeStruct((n, d), jnp.float32),
                jax.ShapeDtypeStruct((_G, d), jnp.float32),
            ],
        )

    def body(hp0, hp1, invc_ref, b_ref, gna_ref):
        i = pl.program_id(0)
        h = (hp0[0] + hp1[0]) * invc_ref[...]
        common(i, h, b_ref, gna_ref, None)

    return pl.pallas_call(
        body,
        grid=grid,
        in_specs=[blk_p, blk_p1, blk_n, blk_b],
        out_specs=[blk_g],
        out_shape=[jax.ShapeDtypeStruct((_G, d), jnp.float32)],
    )


def _mk_bd(w, in_step, out_step, reps, in_base, out_base):
    m = jnp.zeros((128, 128), jnp.float32)
    for t in range(reps):
        m = m.at[in_base + t * in_step:in_base + t * in_step + w.shape[0],
                 out_base + t * out_step:out_base + t * out_step + w.shape[1]
                 ].set(w)
    return m


def _mlp_build(e_pad, bep, d, hid):
    rows = e_pad // 8
    grid = (rows // bep,)
    blk_e = pl.BlockSpec((bep, 128), lambda i: (i, 0))
    blk_m = pl.BlockSpec((128, 128), lambda i: (0, 0))
    blk_b = pl.BlockSpec((1, 128), lambda i: (0, 0))

    def body(hr_ref, ea_ref, mae, mao, mbe, mbo, m2e, m2o, b1p, b2p, e_ref):
        hr = hr_ref[...]
        ea = ea_ref[...]
        ze = (jnp.dot(hr, mae[...], preferred_element_type=jnp.float32)
              + jnp.dot(ea, mbe[...], preferred_element_type=jnp.float32)
              + b1p[...])
        zo = (jnp.dot(hr, mao[...], preferred_element_type=jnp.float32)
              + jnp.dot(ea, mbo[...], preferred_element_type=jnp.float32)
              + b1p[...])
        e_ref[...] = (
            jnp.dot(jnp.maximum(ze, 0.0), m2e[...],
                    preferred_element_type=jnp.float32)
            + jnp.dot(jnp.maximum(zo, 0.0), m2o[...],
                      preferred_element_type=jnp.float32)
            + b2p[...])

    return pl.pallas_call(
        body,
        grid=grid,
        in_specs=[blk_e, blk_e] + [blk_m] * 6 + [blk_b] * 2,
        out_specs=blk_e,
        out_shape=jax.ShapeDtypeStruct((rows, 128), jnp.float32),
    )


def _mlp_mats(w1, b1, w2, b2, d):
    w1a, w1b = w1[:d], w1[d:]
    return (_mk_bd(w1a, 16, 32, 4, 0, 0), _mk_bd(w1a, 16, 32, 4, 64, 0),
            _mk_bd(w1b, 16, 32, 4, 0, 0), _mk_bd(w1b, 16, 32, 4, 64, 0),
            _mk_bd(w2, 32, 16, 4, 0, 0), _mk_bd(w2, 32, 16, 4, 0, 64),
            jnp.tile(b1, 4).reshape(1, 128), jnp.tile(b2, 8).reshape(1, 128))


def _final_build(d, hid, out_dim):
    def body(gna0, gna1, gna2, ncnt, gea0, gea1, gea2, ecnt,
             w1, b1, w2, b2, w3, b3, an_ref, ae_ref, o_ref):
        ninv = 1.0 / jnp.maximum(ncnt[...], 1.0)
        einv = 1.0 / jnp.maximum(ecnt[0] + ecnt[1], 1.0)
        an = jnp.concatenate(
            [gna0[...] * ninv, gna1[...] * ninv, gna2[...] * ninv], axis=1)
        ae = jnp.concatenate(
            [(gea0[0] + gea0[1]) * einv, (gea1[0] + gea1[1]) * einv,
             (gea2[0] + gea2[1]) * einv], axis=1)
        att = jnp.concatenate([an, ae], axis=1)
        o = jnp.maximum(jnp.dot(att, w1[...], preferred_element_type=jnp.float32)
                        + b1[...], 0.0)
        o = jnp.maximum(jnp.dot(o, w2[...], preferred_element_type=jnp.float32)
                        + b2[...], 0.0)
        o = jnp.dot(o, w3[...], preferred_element_type=jnp.float32) + b3[...]
        an_ref[...] = an
        ae_ref[...] = ae
        o_ref[...] = o

    return pl.pallas_call(
        body,
        out_shape=[
            jax.ShapeDtypeStruct((_G, 3 * d), jnp.float32),
            jax.ShapeDtypeStruct((_G, 3 * d), jnp.float32),
            jax.ShapeDtypeStruct((_G, out_dim), jnp.float32),
        ],
    )


def kernel(x, edge_index, edge_attr, node_weight, batch, edge_attr_batch,
           l0_w1, l0_b1, l0_w2, l0_b2, l1_w1, l1_b1, l1_w2, l1_b2,
           out_w1, out_b1, out_w2, out_b2, out_w3, out_b3):
    n, d = x.shape
    e = edge_index.shape[1]
    hid = l0_w1.shape[1]
    out_dim = out_w3.shape[1]

    blk = _NW * _CH_MAX
    e_pad = -(-e // blk) * blk
    n_stage_sc = e_pad // (_NW * _SUB * _NSUB_SC)
    n_stage_g = e_pad // (_NW * _SUB * _NSUB_G)
    pad = e_pad - e
    pool_base = n + 8
    np_rows = -(-(pool_base + _G + 1) // (_NS * 8)) * (_NS * 8)

    row = edge_index[0]
    col = edge_index[1]
    colp = jnp.concatenate([col, jnp.full((pad,), n, jnp.int32)])
    rowp = jnp.concatenate([row, jnp.zeros((pad,), jnp.int32)])
    ea_pk = jnp.concatenate(
        [edge_attr.reshape(e // 8, 8 * d),
         jnp.zeros((pad // 8, 8 * d), jnp.float32)])
    eap = ea_pk.reshape(e_pad, d)
    ebp = jnp.concatenate([edge_attr_batch + pool_base,
                           jnp.full((pad,), pool_base + _G, jnp.int32)])
    col2 = colp.reshape(e_pad // _SUB, _SUB)
    row2 = rowp.reshape(e_pad // _SUB, _SUB)
    gb2 = ebp.reshape(e_pad // _SUB, _SUB)
    bn = 4000
    bep = 4096
    batch2 = batch.reshape(n // bn, 1, bn)

    scatter_cnt = _sc_scatter_build(e_pad, np_rows, n_stage_sc, d, True)
    scatter = _sc_scatter_build(e_pad, np_rows, n_stage_sc, d, False)
    gather = _sc_gather_build(e_pad, n, n_stage_g, d)
    fin0 = _finalize_build(n, np_rows, bn, d, 0)
    fin1 = _finalize_build(n, np_rows, bn, d, 1)
    fin2 = _finalize_build(n, np_rows, bn, d, 2)
    mlp = _mlp_build(e_pad, bep, d, hid)
    final = _final_build(d, hid, out_dim)

    eap_p = ea_pk

    hp, cp = scatter_cnt(eap, col2, gb2)
    h0, invc, gna0, ncnt = fin0(hp, hp, cp, cp, batch2)
    hrow0 = gather(h0, row2)
    e0_p = mlp(hrow0.reshape(e_pad // 8, 8 * d), eap_p,
               *_mlp_mats(l0_w1, l0_b1, l0_w2, l0_b2, d))
    e0 = e0_p.reshape(e_pad, d)
    hp1 = scatter(e0, col2, gb2)
    h1, gna1 = fin1(hp1, hp1, invc, batch2)
    hrow1 = gather(h1, row2)
    e1 = mlp(hrow1.reshape(e_pad // 8, 8 * d), e0_p,
             *_mlp_mats(l1_w1, l1_b1, l1_w2, l1_b2, d)).reshape(e_pad, d)
    hp2 = scatter(e1, col2, gb2)
    (gna2,) = fin2(hp2, hp2, invc, batch2)

    gea0 = lax.slice(hp, (0, pool_base, 0), (2, pool_base + _G, d))
    gea1 = lax.slice(hp1, (0, pool_base, 0), (2, pool_base + _G, d))
    gea2 = lax.slice(hp2, (0, pool_base, 0), (2, pool_base + _G, d))
    ecnt = lax.slice(cp, (0, pool_base, 0), (2, pool_base + _G, d))

    all_node, all_edge, o = final(
        gna0, gna1, gna2, ncnt, gea0, gea1, gea2, ecnt,
        out_w1, out_b1.reshape(1, -1), out_w2, out_b2.reshape(1, -1),
        out_w3, out_b3.reshape(1, -1))
    return (all_node, all_edge, o)


# speedup vs baseline: 10.3970x; 1.0078x over previous
import functools

import jax
import jax.numpy as jnp
from jax import lax
from jax.experimental import pallas as pl
from jax.experimental.pallas import tpu as pltpu
from jax.experimental.pallas import tpu_sc as plsc

_NC = 2
_NS = 16
_NW = _NC * _NS
_LANES = 16

_SUB = 128
_NSUB_SC = 10
_NSUB_G = 20
_CH_MAX = _SUB * _NSUB_G * 2

_G = 16


def _mesh():
    return plsc.VectorSubcoreMesh(core_axis_name="c", subcore_axis_name="s")


def _sc_scatter_build(e_pad, np_rows, n_stage, d, with_count):
    nsub = _NSUB_SC
    ch = _SUB * nsub
    tpr = np_rows // _NS
    nfull = tpr // ch
    rem = tpr - nfull * ch

    out_types = [jax.ShapeDtypeStruct((_NC, np_rows, d), jnp.float32)]
    if with_count:
        out_types.append(jax.ShapeDtypeStruct((_NC, np_rows, d), jnp.float32))

    def body(src_hbm, idx_hbm, gidx_hbm, *rest):
        if with_count:
            hout, cout, rows_v, idx_v, gidx_v, acc, sem = rest
        else:
            hout, rows_v, idx_v, gidx_v, acc, sem = rest
            cout = None
        cid = lax.axis_index("c")
        sid = lax.axis_index("s")
        wid = sid * _NC + cid
        base = sid * tpr

        rows_r = rows_v

        def zero_rows():
            @pl.loop(0, ch)
            def _z(i):
                rows_v[i] = jnp.zeros((_LANES,), jnp.float32)

        def zero_acc():
            for k in range(nfull):
                pltpu.sync_copy(rows_r, acc.at[pl.ds(base + k * ch, ch)])
            if rem:
                pltpu.sync_copy(rows_r.at[pl.ds(0, rem)],
                                acc.at[pl.ds(base + nfull * ch, rem)])

        def drain(dst):
            for k in range(nfull):
                pltpu.sync_copy(acc.at[pl.ds(base + k * ch, ch)], rows_r)
                pltpu.sync_copy(rows_r, dst.at[cid, pl.ds(base + k * ch, ch)])
            if rem:
                pltpu.sync_copy(acc.at[pl.ds(base + nfull * ch, rem)],
                                rows_r.at[pl.ds(0, rem)])
                pltpu.sync_copy(rows_r.at[pl.ds(0, rem)],
                                dst.at[cid, pl.ds(base + nfull * ch, rem)])

        def scatter_pass(idx2, stage_rows):
            @pl.loop(0, n_stage)
            def _stage(s):
                st = wid * n_stage + s
                i_cp = pltpu.async_copy(idx_hbm.at[pl.ds(st * nsub, nsub)],
                                        idx_v, sem)
                g_cp = pltpu.async_copy(gidx_hbm.at[pl.ds(st * nsub, nsub)],
                                        gidx_v, sem)
                stage_rows(st)
                i_cp.wait()
                g_cp.wait()
                descs = []
                for j in range(nsub):
                    src = rows_r.at[pl.ds(j * _SUB if idx2 else 0, _SUB)]
                    descs.append(pltpu.async_copy(
                        src, acc.at[idx_v.at[j]], sem, add=True))
                    descs.append(pltpu.async_copy(
                        src, acc.at[gidx_v.at[j]], sem, add=True))
                for dsc in descs:
                    dsc.wait()

        zero_rows()
        zero_acc()
        plsc.subcore_barrier()

        def stage_rows_main(st):
            pltpu.sync_copy(src_hbm.at[pl.ds(st * ch, ch)], rows_v)

        scatter_pass(True, stage_rows_main)

        plsc.subcore_barrier()
        drain(hout)

        if with_count:
            plsc.subcore_barrier()
            zero_rows()
            zero_acc()
            plsc.subcore_barrier()

            @pl.loop(0, _SUB)
            def _ones(i):
                rows_v[i] = jnp.ones((_LANES,), jnp.float32)

            scatter_pass(False, lambda st: None)

            plsc.subcore_barrier()
            drain(cout)

    return pl.kernel(
        body,
        out_type=tuple(out_types) if with_count else out_types[0],
        mesh=_mesh(),
        compiler_params=pltpu.CompilerParams(use_tc_tiling_on_sc=False),
        scratch_types=[
            pltpu.VMEM((ch, d), jnp.float32),
            pltpu.VMEM((nsub, _SUB), jnp.int32),
            pltpu.VMEM((nsub, _SUB), jnp.int32),
            pltpu.VMEM_SHARED((np_rows, d), jnp.float32),
            pltpu.SemaphoreType.DMA,
        ],
    )


def _sc_gather_build(e_pad, n_src, n_stage, d):
    nsub = _NSUB_G
    ch = _SUB * nsub
    assert n_stage >= 2 and n_stage % 2 == 0

    def body(tab_hbm, idx_hbm, out_hbm, rows_a, rows_b, idx_a, idx_b,
             semg, semo_a, semo_b):
        cid = lax.axis_index("c")
        sid = lax.axis_index("s")
        wid = sid * _NC + cid
        t0 = wid * n_stage

        def do_slot(st, rows_v, idx_v, semo):
            pltpu.sync_copy(idx_hbm.at[pl.ds(st * nsub, nsub)], idx_v)
            descs = [pltpu.async_copy(tab_hbm.at[idx_v.at[j]],
                                      rows_v.at[pl.ds(j * _SUB, _SUB)], semg)
                     for j in range(nsub)]
            for dsc in descs:
                dsc.wait()
            pltpu.async_copy(rows_v, out_hbm.at[pl.ds(st * ch, ch)], semo)

        def wait_out(rows_v, semo):
            pltpu.make_async_copy(rows_v, out_hbm.at[pl.ds(0, ch)],
                                  semo).wait()

        do_slot(t0, rows_a, idx_a, semo_a)
        do_slot(t0 + 1, rows_b, idx_b, semo_b)

        @pl.loop(2, n_stage, step=2)
        def _pair(s0):
            wait_out(rows_a, semo_a)
            do_slot(t0 + s0, rows_a, idx_a, semo_a)
            wait_out(rows_b, semo_b)
            do_slot(t0 + s0 + 1, rows_b, idx_b, semo_b)

        wait_out(rows_a, semo_a)
        wait_out(rows_b, semo_b)

    return pl.kernel(
        body,
        out_type=jax.ShapeDtypeStruct((e_pad, d), jnp.float32),
        mesh=_mesh(),
        compiler_params=pltpu.CompilerParams(use_tc_tiling_on_sc=False),
        scratch_types=[
            pltpu.VMEM((ch, d), jnp.float32),
            pltpu.VMEM((ch, d), jnp.float32),
            pltpu.VMEM((nsub, _SUB), jnp.int32),
            pltpu.VMEM((nsub, _SUB), jnp.int32),
            pltpu.SemaphoreType.DMA,
            pltpu.SemaphoreType.DMA,
            pltpu.SemaphoreType.DMA,
        ],
    )


def _onehot_t(ids_row, bn):
    io = lax.broadcasted_iota(jnp.int32, (_G, bn), 0)
    return (ids_row == io).astype(jnp.float32)


def _finalize_build(n, np_rows, bn, d, mode):
    grid = (n // bn,)
    blk_p = pl.BlockSpec((1, bn, d), lambda i: (0, i, 0))
    blk_p1 = pl.BlockSpec((1, bn, d), lambda i: (1, i, 0))
    blk_n = pl.BlockSpec((bn, d), lambda i: (i, 0))
    blk_b = pl.BlockSpec((1, 1, bn), lambda i: (i, 0, 0))
    blk_g = pl.BlockSpec((_G, d), lambda i: (0, 0))

    def common(i, h, b_ref, gna_ref, ncnt_ref):
        oht = _onehot_t(b_ref[0], bn)

        @pl.when(i == 0)
        def _():
            gna_ref[...] = jnp.zeros_like(gna_ref)
            if ncnt_ref is not None:
                ncnt_ref[...] = jnp.zeros_like(ncnt_ref)

        gna_ref[...] += jnp.dot(oht, h, preferred_element_type=jnp.float32)
        if ncnt_ref is not None:
            ncnt_ref[...] += jnp.dot(oht, jnp.ones((bn, d), jnp.float32),
                                     preferred_element_type=jnp.float32)

    if mode == 0:
        def body(hp0, hp1, cp0, cp1, b_ref, h_ref, invc_ref, gna_ref, ncnt_ref):
            i = pl.program_id(0)
            p = hp0[0] + hp1[0]
            c = jnp.maximum(cp0[0] + cp1[0], 1.0)
            invc = 1.0 / c
            h = p * invc
            h_ref[...] = h
            invc_ref[...] = invc
            common(i, h, b_ref, gna_ref, ncnt_ref)

        return pl.pallas_call(
            body,
            grid=grid,
            in_specs=[blk_p, blk_p1, blk_p, blk_p1, blk_b],
            out_specs=[blk_n, blk_n, blk_g, blk_g],
            out_shape=[
                jax.ShapeDtypeStruct((n, d), jnp.float32),
                jax.ShapeDtypeStruct((n, d), jnp.float32),
                jax.ShapeDtypeStruct((_G, d), jnp.float32),
                jax.ShapeDtypeStruct((_G, d), jnp.float32),
            ],
        )

    if mode == 1:
        def body(hp0, hp1, invc_ref, b_ref, h_ref, gna_ref):
            i = pl.program_id(0)
            h = (hp0[0] + hp1[0]) * invc_ref[...]
            h_ref[...] = h
            common(i, h, b_ref, gna_ref, None)

        return pl.pallas_call(
            body,
            grid=grid,
            in_specs=[blk_p, blk_p1, blk_n, blk_b],
            out_specs=[blk_n, blk_g],
            out_shape=[
                jax.ShapeDtypeStruct((n, d), jnp.float32),
                jax.ShapeDtypeStruct((_G, d), jnp.float32),
            ],
        )

    def body(hp0, hp1, invc_ref, b_ref, gna_ref):
        i = pl.program_id(0)
        h = (hp0[0] + hp1[0]) * invc_ref[...]
        common(i, h, b_ref, gna_ref, None)

    return pl.pallas_call(
        body,
        grid=grid,
        in_specs=[blk_p, blk_p1, blk_n, blk_b],
        out_specs=[blk_g],
        out_shape=[jax.ShapeDtypeStruct((_G, d), jnp.float32)],
    )


def _mk_bd(w, in_step, out_step, reps, in_base, out_base):
    m = jnp.zeros((128, 128), jnp.float32)
    for t in range(reps):
        m = m.at[in_base + t * in_step:in_base + t * in_step + w.shape[0],
                 out_base + t * out_step:out_base + t * out_step + w.shape[1]
                 ].set(w)
    return m


def _mlp_build(e_pad, bep, d, hid):
    rows = e_pad // 8
    grid = (rows // bep,)
    blk_e = pl.BlockSpec((bep, 128), lambda i: (i, 0))
    blk_m = pl.BlockSpec((128, 128), lambda i: (0, 0))
    blk_b = pl.BlockSpec((1, 128), lambda i: (0, 0))

    def body(hr_ref, ea_ref, mae, mao, mbe, mbo, m2e, m2o, b1p, b2p, e_ref):
        hr = hr_ref[...]
        ea = ea_ref[...]
        ze = (jnp.dot(hr, mae[...], preferred_element_type=jnp.float32)
              + jnp.dot(ea, mbe[...], preferred_element_type=jnp.float32)
              + b1p[...])
        zo = (jnp.dot(hr, mao[...], preferred_element_type=jnp.float32)
              + jnp.dot(ea, mbo[...], preferred_element_type=jnp.float32)
              + b1p[...])
        e_ref[...] = (
            jnp.dot(jnp.maximum(ze, 0.0), m2e[...],
                    preferred_element_type=jnp.float32)
            + jnp.dot(jnp.maximum(zo, 0.0), m2o[...],
                      preferred_element_type=jnp.float32)
            + b2p[...])

    return pl.pallas_call(
        body,
        grid=grid,
        in_specs=[blk_e, blk_e] + [blk_m] * 6 + [blk_b] * 2,
        out_specs=blk_e,
        out_shape=jax.ShapeDtypeStruct((rows, 128), jnp.float32),
    )


def _mlp_mats(w1, b1, w2, b2, d):
    w1a, w1b = w1[:d], w1[d:]
    return (_mk_bd(w1a, 16, 32, 4, 0, 0), _mk_bd(w1a, 16, 32, 4, 64, 0),
            _mk_bd(w1b, 16, 32, 4, 0, 0), _mk_bd(w1b, 16, 32, 4, 64, 0),
            _mk_bd(w2, 32, 16, 4, 0, 0), _mk_bd(w2, 32, 16, 4, 0, 64),
            jnp.tile(b1, 4).reshape(1, 128), jnp.tile(b2, 8).reshape(1, 128))


def _final_build(d, hid, out_dim):
    def body(gna0, gna1, gna2, ncnt, gea0, gea1, gea2, ecnt,
             w1, b1, w2, b2, w3, b3, an_ref, ae_ref, o_ref):
        ninv = 1.0 / jnp.maximum(ncnt[...], 1.0)
        einv = 1.0 / jnp.maximum(ecnt[0] + ecnt[1], 1.0)
        an = jnp.concatenate(
            [gna0[...] * ninv, gna1[...] * ninv, gna2[...] * ninv], axis=1)
        ae = jnp.concatenate(
            [(gea0[0] + gea0[1]) * einv, (gea1[0] + gea1[1]) * einv,
             (gea2[0] + gea2[1]) * einv], axis=1)
        att = jnp.concatenate([an, ae], axis=1)
        o = jnp.maximum(jnp.dot(att, w1[...], preferred_element_type=jnp.float32)
                        + b1[...], 0.0)
        o = jnp.maximum(jnp.dot(o, w2[...], preferred_element_type=jnp.float32)
                        + b2[...], 0.0)
        o = jnp.dot(o, w3[...], preferred_element_type=jnp.float32) + b3[...]
        an_ref[...] = an
        ae_ref[...] = ae
        o_ref[...] = o

    return pl.pallas_call(
        body,
        out_shape=[
            jax.ShapeDtypeStruct((_G, 3 * d), jnp.float32),
            jax.ShapeDtypeStruct((_G, 3 * d), jnp.float32),
            jax.ShapeDtypeStruct((_G, out_dim), jnp.float32),
        ],
    )


def kernel(x, edge_index, edge_attr, node_weight, batch, edge_attr_batch,
           l0_w1, l0_b1, l0_w2, l0_b2, l1_w1, l1_b1, l1_w2, l1_b2,
           out_w1, out_b1, out_w2, out_b2, out_w3, out_b3):
    n, d = x.shape
    e = edge_index.shape[1]
    hid = l0_w1.shape[1]
    out_dim = out_w3.shape[1]

    blk = _NW * _CH_MAX
    e_pad = -(-e // blk) * blk
    n_stage_sc = e_pad // (_NW * _SUB * _NSUB_SC)
    n_stage_g = e_pad // (_NW * _SUB * _NSUB_G)
    pad = e_pad - e
    pool_base = n + 8
    np_rows = -(-(pool_base + _G + 1) // (_NS * 8)) * (_NS * 8)

    row = edge_index[0]
    col = edge_index[1]
    colp = jnp.concatenate([col, jnp.full((pad,), n, jnp.int32)])
    rowp = jnp.concatenate([row, jnp.zeros((pad,), jnp.int32)])
    ea_pk = jnp.concatenate(
        [edge_attr.reshape(e // 8, 8 * d),
         jnp.zeros((pad // 8, 8 * d), jnp.float32)])
    eap = ea_pk.reshape(e_pad, d)
    ebp = jnp.concatenate([edge_attr_batch + pool_base,
                           jnp.full((pad,), pool_base + _G, jnp.int32)])
    col2 = colp.reshape(e_pad // _SUB, _SUB)
    row2 = rowp.reshape(e_pad // _SUB, _SUB)
    gb2 = ebp.reshape(e_pad // _SUB, _SUB)
    bn = 4000
    bep = 4096
    batch2 = batch.reshape(n // bn, 1, bn)

    scatter_cnt = _sc_scatter_build(e_pad, np_rows, n_stage_sc, d, True)
    scatter = _sc_scatter_build(e_pad, np_rows, n_stage_sc, d, False)
    gather = _sc_gather_build(e_pad, n, n_stage_g, d)
    fin0 = _finalize_build(n, np_rows, bn, d, 0)
    fin1 = _finalize_build(n, np_rows, bn, d, 1)
    fin2 = _finalize_build(n, np_rows, bn, d, 2)
    mlp = _mlp_build(e_pad, bep, d, hid)
    final = _final_build(d, hid, out_dim)

    eap_p = ea_pk

    hp, cp = scatter_cnt(eap, col2, gb2)
    h0, invc, gna0, ncnt = fin0(hp, hp, cp, cp, batch2)
    hrow0 = gather(h0, row2)
    e0_p = mlp(hrow0.reshape(e_pad // 8, 8 * d), eap_p,
               *_mlp_mats(l0_w1, l0_b1, l0_w2, l0_b2, d))
    e0 = e0_p.reshape(e_pad, d)
    hp1 = scatter(e0, col2, gb2)
    h1, gna1 = fin1(hp1, hp1, invc, batch2)
    hrow1 = gather(h1, row2)
    e1 = mlp(hrow1.reshape(e_pad // 8, 8 * d), e0_p,
             *_mlp_mats(l1_w1, l1_b1, l1_w2, l1_b2, d)).reshape(e_pad, d)
    hp2 = scatter(e1, col2, gb2)
    (gna2,) = fin2(hp2, hp2, invc, batch2)

    gea0 = lax.slice(hp, (0, pool_base, 0), (2, pool_base + _G, d))
    gea1 = lax.slice(hp1, (0, pool_base, 0), (2, pool_base + _G, d))
    gea2 = lax.slice(hp2, (0, pool_base, 0), (2, pool_base + _G, d))
    ecnt = lax.slice(cp, (0, pool_base, 0), (2, pool_base + _G, d))

    all_node, all_edge, o = final(
        gna0, gna1, gna2, ncnt, gea0, gea1, gea2, ecnt,
        out_w1, out_b1.reshape(1, -1), out_w2, out_b2.reshape(1, -1),
        out_w3, out_b3.reshape(1, -1))
    return (all_node, all_edge, o)


# speedup vs baseline: 11.2139x; 1.0786x over previous
import functools

import jax
import jax.numpy as jnp
from jax import lax
from jax.experimental import pallas as pl
from jax.experimental.pallas import tpu as pltpu
from jax.experimental.pallas import tpu_sc as plsc

_NC = 2
_NS = 16
_NW = _NC * _NS
_LANES = 16

_SUB = 128
_NSUB_SC = 10
_NSUB_G = 20
_CH_MAX = _SUB * _NSUB_G * 2

_G = 16


def _mesh():
    return plsc.VectorSubcoreMesh(core_axis_name="c", subcore_axis_name="s")


def _sc_scatter_build(e_pad, np_rows, n_stage, d, with_count):
    nsub = _NSUB_SC
    ch = _SUB * nsub
    tpr = np_rows // _NS
    nfull = tpr // ch
    rem = tpr - nfull * ch

    def body(*args):
        if with_count:
            idx_hbm, gidx_hbm, hout, rows_v, idx_v, gidx_v, acc, sem = args
            src_hbm = None
        else:
            (src_hbm, idx_hbm, gidx_hbm, hout,
             rows_v, idx_v, gidx_v, acc, sem) = args
        cid = lax.axis_index("c")
        sid = lax.axis_index("s")
        wid = sid * _NC + cid
        base = sid * tpr

        rows_r = rows_v

        def zero_rows():
            @pl.loop(0, ch)
            def _z(i):
                rows_v[i] = jnp.zeros((_LANES,), jnp.float32)

        def zero_acc():
            for k in range(nfull):
                pltpu.sync_copy(rows_r, acc.at[pl.ds(base + k * ch, ch)])
            if rem:
                pltpu.sync_copy(rows_r.at[pl.ds(0, rem)],
                                acc.at[pl.ds(base + nfull * ch, rem)])

        def drain(dst):
            for k in range(nfull):
                pltpu.sync_copy(acc.at[pl.ds(base + k * ch, ch)], rows_r)
                pltpu.sync_copy(rows_r, dst.at[cid, pl.ds(base + k * ch, ch)])
            if rem:
                pltpu.sync_copy(acc.at[pl.ds(base + nfull * ch, rem)],
                                rows_r.at[pl.ds(0, rem)])
                pltpu.sync_copy(rows_r.at[pl.ds(0, rem)],
                                dst.at[cid, pl.ds(base + nfull * ch, rem)])

        def scatter_pass(idx2, stage_rows):
            @pl.loop(0, n_stage)
            def _stage(s):
                st = wid * n_stage + s
                i_cp = pltpu.async_copy(idx_hbm.at[pl.ds(st * nsub, nsub)],
                                        idx_v, sem)
                g_cp = pltpu.async_copy(gidx_hbm.at[pl.ds(st * nsub, nsub)],
                                        gidx_v, sem)
                stage_rows(st)
                i_cp.wait()
                g_cp.wait()
                descs = []
                for j in range(nsub):
                    src = rows_r.at[pl.ds(j * _SUB if idx2 else 0, _SUB)]
                    descs.append(pltpu.async_copy(
                        src, acc.at[idx_v.at[j]], sem, add=True))
                    descs.append(pltpu.async_copy(
                        src, acc.at[gidx_v.at[j]], sem, add=True))
                for dsc in descs:
                    dsc.wait()

        zero_rows()
        zero_acc()
        plsc.subcore_barrier()

        if with_count:
            @pl.loop(0, _SUB)
            def _ones(i):
                rows_v[i] = jnp.ones((_LANES,), jnp.float32)

            scatter_pass(False, lambda st: None)
        else:
            def stage_rows_main(st):
                pltpu.sync_copy(src_hbm.at[pl.ds(st * ch, ch)], rows_v)

            scatter_pass(True, stage_rows_main)

        plsc.subcore_barrier()
        drain(hout)

    return pl.kernel(
        body,
        out_type=jax.ShapeDtypeStruct((_NC, np_rows, d), jnp.float32),
        mesh=_mesh(),
        compiler_params=pltpu.CompilerParams(use_tc_tiling_on_sc=False),
        scratch_types=[
            pltpu.VMEM((ch, d), jnp.float32),
            pltpu.VMEM((nsub, _SUB), jnp.int32),
            pltpu.VMEM((nsub, _SUB), jnp.int32),
            pltpu.VMEM_SHARED((np_rows, d), jnp.float32),
            pltpu.SemaphoreType.DMA,
        ],
    )


def _sc_gather_build(e_pad, n_src, n_stage, d):
    nsub = _NSUB_G
    ch = _SUB * nsub
    assert n_stage >= 2 and n_stage % 2 == 0

    def body(tab_hbm, idx_hbm, out_hbm, rows_a, rows_b, idx_a, idx_b,
             semg, semo_a, semo_b):
        cid = lax.axis_index("c")
        sid = lax.axis_index("s")
        wid = sid * _NC + cid
        t0 = wid * n_stage

        def do_slot(st, rows_v, idx_v, semo):
            pltpu.sync_copy(idx_hbm.at[pl.ds(st * nsub, nsub)], idx_v)
            descs = [pltpu.async_copy(tab_hbm.at[idx_v.at[j]],
                                      rows_v.at[pl.ds(j * _SUB, _SUB)], semg)
                     for j in range(nsub)]
            for dsc in descs:
                dsc.wait()
            pltpu.async_copy(rows_v, out_hbm.at[pl.ds(st * ch, ch)], semo)

        def wait_out(rows_v, semo):
            pltpu.make_async_copy(rows_v, out_hbm.at[pl.ds(0, ch)],
                                  semo).wait()

        do_slot(t0, rows_a, idx_a, semo_a)
        do_slot(t0 + 1, rows_b, idx_b, semo_b)

        @pl.loop(2, n_stage, step=2)
        def _pair(s0):
            wait_out(rows_a, semo_a)
            do_slot(t0 + s0, rows_a, idx_a, semo_a)
            wait_out(rows_b, semo_b)
            do_slot(t0 + s0 + 1, rows_b, idx_b, semo_b)

        wait_out(rows_a, semo_a)
        wait_out(rows_b, semo_b)

    return pl.kernel(
        body,
        out_type=jax.ShapeDtypeStruct((e_pad, d), jnp.float32),
        mesh=_mesh(),
        compiler_params=pltpu.CompilerParams(use_tc_tiling_on_sc=False),
        scratch_types=[
            pltpu.VMEM((ch, d), jnp.float32),
            pltpu.VMEM((ch, d), jnp.float32),
            pltpu.VMEM((nsub, _SUB), jnp.int32),
            pltpu.VMEM((nsub, _SUB), jnp.int32),
            pltpu.SemaphoreType.DMA,
            pltpu.SemaphoreType.DMA,
            pltpu.SemaphoreType.DMA,
        ],
    )


def _onehot_t(ids_row, bn):
    io = lax.broadcasted_iota(jnp.int32, (_G, bn), 0)
    return (ids_row == io).astype(jnp.float32)


def _finalize_build(n, np_rows, bn, d, mode):
    grid = (n // bn,)
    blk_p = pl.BlockSpec((1, bn, d), lambda i: (0, i, 0))
    blk_p1 = pl.BlockSpec((1, bn, d), lambda i: (1, i, 0))
    blk_n = pl.BlockSpec((bn, d), lambda i: (i, 0))
    blk_b = pl.BlockSpec((1, 1, bn), lambda i: (i, 0, 0))
    blk_g = pl.BlockSpec((_G, d), lambda i: (0, 0))

    def common(i, h, b_ref, gna_ref, ncnt_ref):
        oht = _onehot_t(b_ref[0], bn)

        @pl.when(i == 0)
        def _():
            gna_ref[...] = jnp.zeros_like(gna_ref)
            if ncnt_ref is not None:
                ncnt_ref[...] = jnp.zeros_like(ncnt_ref)

        gna_ref[...] += jnp.dot(oht, h, preferred_element_type=jnp.float32)
        if ncnt_ref is not None:
            ncnt_ref[...] += jnp.dot(oht, jnp.ones((bn, d), jnp.float32),
                                     preferred_element_type=jnp.float32)

    if mode == 0:
        def body(hp0, hp1, cp0, cp1, b_ref, h_ref, invc_ref, gna_ref, ncnt_ref):
            i = pl.program_id(0)
            p = hp0[0] + hp1[0]
            c = jnp.maximum(cp0[0] + cp1[0], 1.0)
            invc = 1.0 / c
            h = p * invc
            h_ref[...] = h
            invc_ref[...] = invc
            common(i, h, b_ref, gna_ref, ncnt_ref)

        return pl.pallas_call(
            body,
            grid=grid,
            in_specs=[blk_p, blk_p1, blk_p, blk_p1, blk_b],
            out_specs=[blk_n, blk_n, blk_g, blk_g],
            out_shape=[
                jax.ShapeDtypeStruct((n, d), jnp.float32),
                jax.ShapeDtypeStruct((n, d), jnp.float32),
                jax.ShapeDtypeStruct((_G, d), jnp.float32),
                jax.ShapeDtypeStruct((_G, d), jnp.float32),
            ],
        )

    if mode == 1:
        def body(hp0, hp1, invc_ref, b_ref, h_ref, gna_ref):
            i = pl.program_id(0)
            h = (hp0[0] + hp1[0]) * invc_ref[...]
            h_ref[...] = h
            common(i, h, b_ref, gna_ref, None)

        return pl.pallas_call(
            body,
            grid=grid,
            in_specs=[blk_p, blk_p1, blk_n, blk_b],
            out_specs=[blk_n, blk_g],
            out_shape=[
                jax.ShapeDtypeStruct((n, d), jnp.float32),
                jax.ShapeDtypeStruct((_G, d), jnp.float32),
            ],
        )

    def body(hp0, hp1, invc_ref, b_ref, gna_ref):
        i = pl.program_id(0)
        h = (hp0[0] + hp1[0]) * invc_ref[...]
        common(i, h, b_ref, gna_ref, None)

    return pl.pallas_call(
        body,
        grid=grid,
        in_specs=[blk_p, blk_p1, blk_n, blk_b],
        out_specs=[blk_g],
        out_shape=[jax.ShapeDtypeStruct((_G, d), jnp.float32)],
    )


def _mk_bd(w, in_step, out_step, reps, in_base, out_base):
    m = jnp.zeros((128, 128), jnp.float32)
    for t in range(reps):
        m = m.at[in_base + t * in_step:in_base + t * in_step + w.shape[0],
                 out_base + t * out_step:out_base + t * out_step + w.shape[1]
                 ].set(w)
    return m


def _mlp_build(e_pad, bep, d, hid):
    rows = e_pad // 8
    grid = (rows // bep,)
    blk_e = pl.BlockSpec((bep, 128), lambda i: (i, 0))
    blk_m = pl.BlockSpec((128, 128), lambda i: (0, 0))
    blk_b = pl.BlockSpec((1, 128), lambda i: (0, 0))

    def body(hr_ref, ea_ref, mae, mao, mbe, mbo, m2e, m2o, b1p, b2p, e_ref):
        hr = hr_ref[...]
        ea = ea_ref[...]
        ze = (jnp.dot(hr, mae[...], preferred_element_type=jnp.float32)
              + jnp.dot(ea, mbe[...], preferred_element_type=jnp.float32)
              + b1p[...])
        zo = (jnp.dot(hr, mao[...], preferred_element_type=jnp.float32)
              + jnp.dot(ea, mbo[...], preferred_element_type=jnp.float32)
              + b1p[...])
        e_ref[...] = (
            jnp.dot(jnp.maximum(ze, 0.0), m2e[...],
                    preferred_element_type=jnp.float32)
            + jnp.dot(jnp.maximum(zo, 0.0), m2o[...],
                      preferred_element_type=jnp.float32)
            + b2p[...])

    return pl.pallas_call(
        body,
        grid=grid,
        in_specs=[blk_e, blk_e] + [blk_m] * 6 + [blk_b] * 2,
        out_specs=blk_e,
        out_shape=jax.ShapeDtypeStruct((rows, 128), jnp.float32),
    )


def _mlp_mats(w1, b1, w2, b2, d):
    w1a, w1b = w1[:d], w1[d:]
    return (_mk_bd(w1a, 16, 32, 4, 0, 0), _mk_bd(w1a, 16, 32, 4, 64, 0),
            _mk_bd(w1b, 16, 32, 4, 0, 0), _mk_bd(w1b, 16, 32, 4, 64, 0),
            _mk_bd(w2, 32, 16, 4, 0, 0), _mk_bd(w2, 32, 16, 4, 0, 64),
            jnp.tile(b1, 4).reshape(1, 128), jnp.tile(b2, 8).reshape(1, 128))


def _final_build(d, hid, out_dim):
    def body(gna0, gna1, gna2, ncnt, gea0, gea1, gea2, ecnt,
             w1, b1, w2, b2, w3, b3, an_ref, ae_ref, o_ref):
        ninv = 1.0 / jnp.maximum(ncnt[...], 1.0)
        einv = 1.0 / jnp.maximum(ecnt[0] + ecnt[1], 1.0)
        an = jnp.concatenate(
            [gna0[...] * ninv, gna1[...] * ninv, gna2[...] * ninv], axis=1)
        ae = jnp.concatenate(
            [(gea0[0] + gea0[1]) * einv, (gea1[0] + gea1[1]) * einv,
             (gea2[0] + gea2[1]) * einv], axis=1)
        att = jnp.concatenate([an, ae], axis=1)
        o = jnp.maximum(jnp.dot(att, w1[...], preferred_element_type=jnp.float32)
                        + b1[...], 0.0)
        o = jnp.maximum(jnp.dot(o, w2[...], preferred_element_type=jnp.float32)
                        + b2[...], 0.0)
        o = jnp.dot(o, w3[...], preferred_element_type=jnp.float32) + b3[...]
        an_ref[...] = an
        ae_ref[...] = ae
        o_ref[...] = o

    return pl.pallas_call(
        body,
        out_shape=[
            jax.ShapeDtypeStruct((_G, 3 * d), jnp.float32),
            jax.ShapeDtypeStruct((_G, 3 * d), jnp.float32),
            jax.ShapeDtypeStruct((_G, out_dim), jnp.float32),
        ],
    )


def kernel(x, edge_index, edge_attr, node_weight, batch, edge_attr_batch,
           l0_w1, l0_b1, l0_w2, l0_b2, l1_w1, l1_b1, l1_w2, l1_b2,
           out_w1, out_b1, out_w2, out_b2, out_w3, out_b3):
    n, d = x.shape
    e = edge_index.shape[1]
    hid = l0_w1.shape[1]
    out_dim = out_w3.shape[1]

    blk = _NW * _CH_MAX
    e_pad = -(-e // blk) * blk
    n_stage_sc = e_pad // (_NW * _SUB * _NSUB_SC)
    n_stage_g = e_pad // (_NW * _SUB * _NSUB_G)
    pad = e_pad - e
    pool_base = n + 8
    np_rows = -(-(pool_base + _G + 1) // (_NS * 8)) * (_NS * 8)

    row = edge_index[0]
    col = edge_index[1]
    colp = jnp.concatenate([col, jnp.full((pad,), n, jnp.int32)])
    rowp = jnp.concatenate([row, jnp.zeros((pad,), jnp.int32)])
    ea_pk = jnp.concatenate(
        [edge_attr.reshape(e // 8, 8 * d),
         jnp.zeros((pad // 8, 8 * d), jnp.float32)])
    eap = ea_pk.reshape(e_pad, d)
    ebp = jnp.concatenate([edge_attr_batch + pool_base,
                           jnp.full((pad,), pool_base + _G, jnp.int32)])
    col2 = colp.reshape(e_pad // _SUB, _SUB)
    row2 = rowp.reshape(e_pad // _SUB, _SUB)
    gb2 = ebp.reshape(e_pad // _SUB, _SUB)
    bn = 4000
    bep = 4096
    batch2 = batch.reshape(n // bn, 1, bn)

    scatter_count = _sc_scatter_build(e_pad, np_rows, n_stage_sc, d, True)
    scatter = _sc_scatter_build(e_pad, np_rows, n_stage_sc, d, False)
    gather = _sc_gather_build(e_pad, n, n_stage_g, d)
    fin0 = _finalize_build(n, np_rows, bn, d, 0)
    fin1 = _finalize_build(n, np_rows, bn, d, 1)
    fin2 = _finalize_build(n, np_rows, bn, d, 2)
    mlp = _mlp_build(e_pad, bep, d, hid)
    final = _final_build(d, hid, out_dim)

    eap_p = ea_pk

    cp = scatter_count(col2, gb2)
    hp = scatter(eap, col2, gb2)
    h0, invc, gna0, ncnt = fin0(hp, hp, cp, cp, batch2)
    hrow0 = gather(h0, row2)
    e0_p = mlp(hrow0.reshape(e_pad // 8, 8 * d), eap_p,
               *_mlp_mats(l0_w1, l0_b1, l0_w2, l0_b2, d))
    e0 = e0_p.reshape(e_pad, d)
    hp1 = scatter(e0, col2, gb2)
    h1, gna1 = fin1(hp1, hp1, invc, batch2)
    hrow1 = gather(h1, row2)
    e1 = mlp(hrow1.reshape(e_pad // 8, 8 * d), e0_p,
             *_mlp_mats(l1_w1, l1_b1, l1_w2, l1_b2, d)).reshape(e_pad, d)
    hp2 = scatter(e1, col2, gb2)
    (gna2,) = fin2(hp2, hp2, invc, batch2)

    gea0 = lax.slice(hp, (0, pool_base, 0), (2, pool_base + _G, d))
    gea1 = lax.slice(hp1, (0, pool_base, 0), (2, pool_base + _G, d))
    gea2 = lax.slice(hp2, (0, pool_base, 0), (2, pool_base + _G, d))
    ecnt = lax.slice(cp, (0, pool_base, 0), (2, pool_base + _G, d))

    all_node, all_edge, o = final(
        gna0, gna1, gna2, ncnt, gea0, gea1, gea2, ecnt,
        out_w1, out_b1.reshape(1, -1), out_w2, out_b2.reshape(1, -1),
        out_w3, out_b3.reshape(1, -1))
    return (all_node, all_edge, o)
